# Initial kernel scaffold; baseline (speedup 1.0000x reference)
import functools

import numpy as np
import jax
import jax.numpy as jnp
from jax import lax
from jax.experimental import pallas as pl
from jax.experimental.pallas import tpu as pltpu
from jax.experimental.pallas import tpu_sc as plsc

N = 10000
E = 320000
NFEAT = 128
NHID = 64
OUT = 64
P = 3

NC = 2
NS = 16
RT = N // NS
K = 80
EPT = E // NS
CH = EPT // K


def _telu(x):
    return x * jnp.tanh(jnp.exp(x))


def _make_spmm(Wh, fuse, x_shared):
    WREG = Wh // 16
    mesh = plsc.VectorSubcoreMesh(core_axis_name="c", subcore_axis_name="s")
    out_type = jax.ShapeDtypeStruct(
        (NC, N, Wh) if fuse else (NC, P, N, Wh), jnp.float32)
    scratch = [
        pltpu.VMEM((K,), jnp.int32),
        pltpu.VMEM((K,), jnp.int32),
        pltpu.VMEM((K,), jnp.float32),
        pltpu.VMEM((K, Wh), jnp.float32),
        pltpu.VMEM((RT, Wh), jnp.float32),
        pltpu.VMEM_SHARED((N, Wh), jnp.float32),
        pltpu.SemaphoreType.DMA,
    ]

    @functools.partial(pl.kernel, out_type=out_type, mesh=mesh,
                       scratch_types=scratch)
    def k(x_hbm, src_hbm, dst_hbm, vals_hbm, out_hbm,
          srcv, dstv, valv, rows, zbuf, acc, sem):
        c = lax.axis_index("c")
        s = lax.axis_index("s")
        ebase = s * EPT
        rbase = s * RT

        zero16 = jnp.zeros((16,), jnp.float32)

        def zb(i, carry):
            for w in range(WREG):
                zbuf[i, pl.ds(w * 16, 16)] = zero16
            return carry

        lax.fori_loop(0, RT, zb, 0)
        pltpu.sync_copy(zbuf, acc.at[pl.ds(rbase, RT)])
        plsc.subcore_barrier()

        for p in range(P):
            xoff = (c * N) if x_shared else ((c * P + p) * N)

            def chunk(i, carry):
                base = ebase + i * K
                pltpu.sync_copy(src_hbm.at[p, pl.ds(base, K)], srcv)
                pltpu.sync_copy(dst_hbm.at[p, pl.ds(base, K)], dstv)
                pltpu.sync_copy(vals_hbm.at[p, pl.ds(base, K)], valv)
                for w in range(K // 16):
                    sl = pl.ds(w * 16, 16)
                    srcv[sl] = srcv[sl] + xoff
                pltpu.async_copy(x_hbm.at[srcv], rows, sem).wait()

                def scale(e, carry2):
                    v = valv[e]
                    for w in range(WREG):
                        sl = pl.ds(w * 16, 16)
                        rows[e, sl] = rows[e, sl] * v
                    return carry2

                lax.fori_loop(0, K, scale, 0)
                pltpu.sync_copy(rows, acc.at[dstv], add=True)
                return carry

            lax.fori_loop(0, CH, chunk, 0)

            if not fuse:
                plsc.subcore_barrier()
                pltpu.sync_copy(acc.at[pl.ds(rbase, RT)],
                                out_hbm.at[c, p, pl.ds(rbase, RT)])
                if p < P - 1:
                    pltpu.sync_copy(zbuf, acc.at[pl.ds(rbase, RT)])
                    plsc.subcore_barrier()
        if fuse:
            plsc.subcore_barrier()
            pltpu.sync_copy(acc.at[pl.ds(rbase, RT)],
                            out_hbm.at[c, pl.ds(rbase, RT)])

    return k


_spmm192 = _make_spmm(96, fuse=False, x_shared=False)
_spmm128 = _make_spmm(64, fuse=False, x_shared=False)
_spmm64 = _make_spmm(32, fuse=True, x_shared=True)


def _tc1(feature, w2cat):
    def body(f_ref, w_ref, o_ref):
        res = jnp.dot(f_ref[...], w_ref[0], preferred_element_type=jnp.float32)
        o_ref[0, 0] = res[:, :96]
        o_ref[1, 0] = res[:, 96:]

    return pl.pallas_call(
        body,
        grid=(P,),
        in_specs=[pl.BlockSpec((N, NFEAT), lambda p: (0, 0)),
                  pl.BlockSpec((1, NFEAT, 192), lambda p: (p, 0, 0))],
        out_specs=pl.BlockSpec((NC, 1, N, 96), lambda p: (0, p, 0, 0)),
        out_shape=jax.ShapeDtypeStruct((NC, P, N, 96), jnp.float32),
    )(feature, w2cat)


def _tc2(s2, b1s, w2s, sh1b, sh2w):
    def body(s_ref, b1_ref, w2_ref, shb_ref, shw_ref, o_ref):
        sspec = s_ref[0, 0, :, :64] + b1_ref[0]
        a = jnp.dot(_telu(sspec), w2_ref[0], preferred_element_type=jnp.float32)
        ssh = jnp.concatenate([s_ref[0, 0, :, 64:96], s_ref[1, 0, :, :32]],
                              axis=1) + shb_ref[...]
        b = jnp.dot(_telu(ssh), shw_ref[...], preferred_element_type=jnp.float32)
        o_ref[0, 0] = a
        o_ref[1, 0] = b

    return pl.pallas_call(
        body,
        grid=(P,),
        in_specs=[pl.BlockSpec((NC, 1, N, 96), lambda p: (0, p, 0, 0)),
                  pl.BlockSpec((1, NHID), lambda p: (p, 0)),
                  pl.BlockSpec((1, NHID, OUT), lambda p: (p, 0, 0)),
                  pl.BlockSpec((1, NHID), lambda p: (0, 0)),
                  pl.BlockSpec((NHID, OUT), lambda p: (0, 0))],
        out_specs=pl.BlockSpec((NC, 1, N, OUT), lambda p: (0, p, 0, 0)),
        out_shape=jax.ShapeDtypeStruct((NC, P, N, OUT), jnp.float32),
    )(s2, b1s, w2s, sh1b, sh2w)


def _tc3(s4, b2s, sh2b, col1w, col1b, col2w, col2b):
    def body(s_ref, b2_ref, shb_ref, c1w_ref, c1b_ref, c2w_ref, c2b_ref,
             spec_ref, shm_ref, hsh_ref, hcol_ref, ps_ref):
        spec = s_ref[0] + b2_ref[:, None, :]
        shm = s_ref[1] + shb_ref[...][None]
        spec_ref[...] = spec
        shm_ref[...] = shm
        hsh_ref[...] = jnp.mean(shm, axis=0)
        rows = []
        for p in range(P):
            sp = spec[p]
            mp = jnp.mean(sp, axis=0)
            mx = jnp.max(sp, axis=0)
            z = jnp.exp(sp - mx[None, :])
            prob = z / jnp.sum(z, axis=0)[None, :]
            ent = -jnp.sum(prob * jnp.log(prob + 1e-06), axis=0)
            rows.append(jnp.concatenate([mp, mx, ent], axis=-1))
        ps_ref[...] = jnp.stack(rows, axis=0)
        concat_sp = jnp.concatenate([spec[0], spec[1], spec[2]], axis=1)
        h = jax.nn.relu(jnp.dot(concat_sp, c1w_ref[...],
                                preferred_element_type=jnp.float32)
                        + c1b_ref[...])
        hcol_ref[...] = jnp.dot(h, c2w_ref[...],
                                preferred_element_type=jnp.float32) + c2b_ref[...]

    return pl.pallas_call(
        body,
        out_shape=[jax.ShapeDtypeStruct((P, N, OUT), jnp.float32),
                   jax.ShapeDtypeStruct((P, N, OUT), jnp.float32),
                   jax.ShapeDtypeStruct((N, OUT), jnp.float32),
                   jax.ShapeDtypeStruct((N, OUT), jnp.float32),
                   jax.ShapeDtypeStruct((P, 3 * OUT), jnp.float32)],
    )(s4, b2s, sh2b, col1w, col1b, col2w, col2b)


def _tc4(spec, r1, vals_all, wt, wp, raw1b, raw2w):
    def body(spec_ref, r1_ref, v_ref, wt_ref, wp_ref, r1b_ref, r2w_ref,
             hsp_ref, u1_ref, v2_ref, v6_ref):
        hsp_ref[...] = jnp.sum(spec_ref[...] * wp_ref[...], axis=0)
        u1 = jnp.sum(r1_ref[...] * wt_ref[...], axis=0) + r1b_ref[...]
        u1_ref[...] = u1
        v = jnp.dot(u1, r2w_ref[...], preferred_element_type=jnp.float32)
        v2_ref[0] = v[:, :32]
        v2_ref[1] = v[:, 32:]
        v6_ref[...] = v_ref[...] * wt_ref[...][:, :, 0]

    return pl.pallas_call(
        body,
        out_shape=[jax.ShapeDtypeStruct((N, OUT), jnp.float32),
                   jax.ShapeDtypeStruct((N, OUT), jnp.float32),
                   jax.ShapeDtypeStruct((NC, N, 32), jnp.float32),
                   jax.ShapeDtypeStruct((P, E), jnp.float32)],
    )(spec, r1, vals_all, wt, wp, raw1b, raw2w)


def _tc5(hsp, hsh, hcol, u1, s6, raw2b, projw, projb):
    def body(hsp_ref, hsh_ref, hcol_ref, u1_ref, s6_ref, r2b_ref,
             pw_ref, pb_ref, out_ref, hraw_ref):
        u2 = jnp.concatenate([s6_ref[0], s6_ref[1]], axis=1) + r2b_ref[...]
        hraw = (u1_ref[...] + u2) * 0.5
        hraw_ref[...] = hraw
        all_feat = jnp.concatenate(
            [hsp_ref[...], hsh_ref[...], hcol_ref[...], hraw], axis=1)
        out_ref[...] = jnp.dot(all_feat, pw_ref[...],
                               preferred_element_type=jnp.float32) + pb_ref[...]

    return pl.pallas_call(
        body,
        out_shape=[jax.ShapeDtypeStruct((N, OUT), jnp.float32),
                   jax.ShapeDtypeStruct((N, OUT), jnp.float32)],
    )(hsp, hsh, hcol, u1, s6, raw2b, projw, projb)


def kernel(feature, edge_index_0, edge_index_1, edge_index_2,
           vals_0, vals_1, vals_2, params):
    src_all = jnp.stack([edge_index_0[1], edge_index_1[1], edge_index_2[1]])
    dst_all = jnp.stack([edge_index_0[0], edge_index_1[0], edge_index_2[0]])
    vals_all = jnp.stack([vals_0, vals_1, vals_2])

    w2cat = jnp.stack([
        jnp.concatenate([params["spec1_W_" + str(p)], params["sh1_W"],
                         params["raw1_W"]], axis=1)
        for p in range(P)])
    x2 = _tc1(feature, w2cat)

    s2 = _spmm192(x2.reshape(NC * P * N, 96), src_all, dst_all, vals_all)

    b1s = jnp.stack([params["spec1_b_" + str(p)] for p in range(P)])
    w2s = jnp.stack([params["spec2_W_" + str(p)] for p in range(P)])
    x4 = _tc2(s2, b1s, w2s, params["sh1_b"].reshape(1, NHID), params["sh2_W"])

    s4 = _spmm128(x4.reshape(NC * P * N, 64), src_all, dst_all, vals_all)

    b2s = jnp.stack([params["spec2_b_" + str(p)] for p in range(P)])
    spec, shm, hsh, hcol, ps = _tc3(
        s4, b2s, params["sh2_b"].reshape(1, OUT), params["col1_W"],
        params["col1_b"].reshape(1, OUT), params["col2_W"],
        params["col2_b"].reshape(1, OUT))

    sim = ps @ ps.T / (np.sqrt(3.0 * OUT) * params["tau"])
    t_mat = jax.nn.softmax(sim, axis=1)
    pi0 = jax.nn.softmax(params["weight_b"].squeeze())
    pi = pi0
    for _ in range(13):
        pi = 0.2 * pi0 + 0.8 * (pi @ t_mat)
    wt = pi.reshape(P, 1, 1)
    wp = jax.nn.softmax(pi).reshape(P, 1, 1)

    r1 = s2[1, :, :, 32:96]
    hsp, u1, v2, vals6 = _tc4(spec, r1, vals_all, wt, wp,
                              params["raw1_b"].reshape(1, OUT),
                              params["raw2_W"])

    s6 = _spmm64(v2.reshape(NC * N, 32), src_all, dst_all, vals6)

    out, hraw = _tc5(hsp, hsh, hcol, u1, s6,
                     params["raw2_b"].reshape(1, OUT), params["proj_W"],
                     params["proj_b"].reshape(1, OUT))

    return (out, spec[0], spec[1], spec[2], shm[0], shm[1], shm[2],
            hcol, hraw)


# speedup vs baseline: 2.6537x; 2.6537x over previous
import functools

import numpy as np
import jax
import jax.numpy as jnp
from jax import lax
from jax.experimental import pallas as pl
from jax.experimental.pallas import tpu as pltpu
from jax.experimental.pallas import tpu_sc as plsc

N = 10000
E = 320000
NFEAT = 128
NHID = 64
OUT = 64
P = 3

NC = 2
NS = 16
NP_ = 10240
RT = NP_ // NS
K = 80
EPT = E // NS
CH = EPT // K


def _telu(x):
    return x * jnp.tanh(jnp.exp(x))


def _make_spmm(Wh, fuse, x_shared):
    WREG = Wh // 16
    mesh = plsc.VectorSubcoreMesh(core_axis_name="c", subcore_axis_name="s")
    out_type = jax.ShapeDtypeStruct(
        (NC, NP_, Wh) if fuse else (NC, P, NP_, Wh), jnp.float32)
    scratch = [
        pltpu.VMEM((K,), jnp.int32),
        pltpu.VMEM((K,), jnp.int32),
        pltpu.VMEM((K,), jnp.float32),
        pltpu.VMEM((K, Wh), jnp.float32),
        pltpu.VMEM((RT, Wh), jnp.float32),
        pltpu.VMEM_SHARED((NP_, Wh), jnp.float32),
        pltpu.SemaphoreType.DMA,
    ]

    @functools.partial(pl.kernel, out_type=out_type, mesh=mesh,
                       scratch_types=scratch,
                       compiler_params=pltpu.CompilerParams(
                           use_tc_tiling_on_sc=False))
    def k(x_hbm, src_hbm, dst_hbm, vals_hbm, out_hbm,
          srcv, dstv, valv, rows, zbuf, acc, sem):
        c = lax.axis_index("c")
        s = lax.axis_index("s")
        ebase = s * EPT
        rbase = s * RT

        zero16 = jnp.zeros((16,), jnp.float32)

        def zb(i, carry):
            for w in range(WREG):
                zbuf[i, pl.ds(w * 16, 16)] = zero16
            return carry

        lax.fori_loop(0, RT, zb, 0)
        pltpu.sync_copy(zbuf, acc.at[pl.ds(rbase, RT)])
        plsc.subcore_barrier()

        for p in range(P):
            xoff = (c * N) if x_shared else ((c * P + p) * N)

            def chunk(i, carry):
                base = p * E + ebase + i * K
                pltpu.sync_copy(src_hbm.at[pl.ds(base, K)], srcv)
                pltpu.sync_copy(dst_hbm.at[pl.ds(base, K)], dstv)
                pltpu.sync_copy(vals_hbm.at[pl.ds(base, K)], valv)
                for w in range(K // 16):
                    sl = pl.ds(w * 16, 16)
                    srcv[sl] = srcv[sl] + xoff
                pltpu.async_copy(x_hbm.at[srcv], rows, sem).wait()

                def scale(g, carry2):
                    vv = valv[pl.ds(g * 16, 16)]
                    for j in range(16):
                        v = vv[j]
                        e = g * 16 + j
                        for w in range(WREG):
                            sl = pl.ds(w * 16, 16)
                            rows[e, sl] = rows[e, sl] * v
                    return carry2

                lax.fori_loop(0, K // 16, scale, 0)
                pltpu.sync_copy(rows, acc.at[dstv], add=True)
                return carry

            lax.fori_loop(0, CH, chunk, 0)

            if not fuse:
                plsc.subcore_barrier()
                pltpu.sync_copy(acc.at[pl.ds(rbase, RT)],
                                out_hbm.at[c, p, pl.ds(rbase, RT)])
                if p < P - 1:
                    pltpu.sync_copy(zbuf, acc.at[pl.ds(rbase, RT)])
                    plsc.subcore_barrier()
        if fuse:
            plsc.subcore_barrier()
            pltpu.sync_copy(acc.at[pl.ds(rbase, RT)],
                            out_hbm.at[c, pl.ds(rbase, RT)])

    return k


_spmm192 = _make_spmm(96, fuse=False, x_shared=False)
_spmm128 = _make_spmm(64, fuse=False, x_shared=False)
_spmm64 = _make_spmm(32, fuse=True, x_shared=True)


def _tc1(feature, w2cat):
    def body(f_ref, w_ref, o_ref):
        res = jnp.dot(f_ref[...], w_ref[0], preferred_element_type=jnp.float32)
        o_ref[0, 0] = res[:, :96]
        o_ref[1, 0] = res[:, 96:]

    return pl.pallas_call(
        body,
        grid=(P,),
        in_specs=[pl.BlockSpec((N, NFEAT), lambda p: (0, 0)),
                  pl.BlockSpec((1, NFEAT, 192), lambda p: (p, 0, 0))],
        out_specs=pl.BlockSpec((NC, 1, N, 96), lambda p: (0, p, 0, 0)),
        out_shape=jax.ShapeDtypeStruct((NC, P, N, 96), jnp.float32),
    )(feature, w2cat)


def _tc2(s2, b1s, w2s, sh1b, sh2w):
    def body(s_ref, b1_ref, w2_ref, shb_ref, shw_ref, o_ref):
        sspec = s_ref[0, 0, :, :64] + b1_ref[0]
        a = jnp.dot(_telu(sspec), w2_ref[0], preferred_element_type=jnp.float32)
        ssh = jnp.concatenate([s_ref[0, 0, :, 64:96], s_ref[1, 0, :, :32]],
                              axis=1) + shb_ref[...]
        b = jnp.dot(_telu(ssh), shw_ref[...], preferred_element_type=jnp.float32)
        o_ref[0, 0] = a
        o_ref[1, 0] = b

    return pl.pallas_call(
        body,
        grid=(P,),
        in_specs=[pl.BlockSpec((NC, 1, N, 96), lambda p: (0, p, 0, 0)),
                  pl.BlockSpec((1, 1, NHID), lambda p: (p, 0, 0)),
                  pl.BlockSpec((1, NHID, OUT), lambda p: (p, 0, 0)),
                  pl.BlockSpec((1, NHID), lambda p: (0, 0)),
                  pl.BlockSpec((NHID, OUT), lambda p: (0, 0))],
        out_specs=pl.BlockSpec((NC, 1, N, OUT), lambda p: (0, p, 0, 0)),
        out_shape=jax.ShapeDtypeStruct((NC, P, N, OUT), jnp.float32),
    )(s2, b1s, w2s, sh1b, sh2w)


def _tc3(s4, b2s, sh2b):
    def body(s_ref, b2_ref, shb_ref, spec_ref, shm_ref, hsh_ref, ps_ref):
        p = pl.program_id(0)
        spec = s_ref[0, 0] + b2_ref[0]
        shm = s_ref[1, 0] + shb_ref[...]
        spec_ref[0] = spec
        shm_ref[0] = shm
        mp = jnp.mean(spec, axis=0)
        mx = jnp.max(spec, axis=0)
        z = jnp.exp(spec - mx[None, :])
        prob = z / jnp.sum(z, axis=0)[None, :]
        ent = -jnp.sum(prob * jnp.log(prob + 1e-06), axis=0)
        ps_ref[0, 0] = jnp.concatenate([mp, mx, ent], axis=-1)

        @pl.when(p == 0)
        def _():
            hsh_ref[...] = shm

        @pl.when(p > 0)
        def _():
            hsh_ref[...] = hsh_ref[...] + shm

        @pl.when(p == P - 1)
        def _():
            hsh_ref[...] = hsh_ref[...] * (1.0 / P)

    return pl.pallas_call(
        body,
        grid=(P,),
        in_specs=[pl.BlockSpec((NC, 1, N, OUT), lambda p: (0, p, 0, 0)),
                  pl.BlockSpec((1, 1, OUT), lambda p: (p, 0, 0)),
                  pl.BlockSpec((1, OUT), lambda p: (0, 0))],
        out_specs=[pl.BlockSpec((1, N, OUT), lambda p: (p, 0, 0)),
                   pl.BlockSpec((1, N, OUT), lambda p: (p, 0, 0)),
                   pl.BlockSpec((N, OUT), lambda p: (0, 0)),
                   pl.BlockSpec((1, 1, 3 * OUT), lambda p: (p, 0, 0))],
        out_shape=[jax.ShapeDtypeStruct((P, N, OUT), jnp.float32),
                   jax.ShapeDtypeStruct((P, N, OUT), jnp.float32),
                   jax.ShapeDtypeStruct((N, OUT), jnp.float32),
                   jax.ShapeDtypeStruct((P, 1, 3 * OUT), jnp.float32)],
    )(s4, b2s, sh2b)


def _tc3b(spec, col1w, col1b, col2w, col2b):
    def body(spec_ref, c1w_ref, c1b_ref, c2w_ref, c2b_ref, hcol_ref, acc_ref):
        p = pl.program_id(0)
        contrib = jnp.dot(spec_ref[0], c1w_ref[0],
                          preferred_element_type=jnp.float32)

        @pl.when(p == 0)
        def _():
            acc_ref[...] = contrib

        @pl.when(p > 0)
        def _():
            acc_ref[...] = acc_ref[...] + contrib

        @pl.when(p == P - 1)
        def _():
            h = jax.nn.relu(acc_ref[...] + c1b_ref[...])
            hcol_ref[...] = jnp.dot(h, c2w_ref[...],
                                    preferred_element_type=jnp.float32) + c2b_ref[...]

    return pl.pallas_call(
        body,
        grid=(P,),
        in_specs=[pl.BlockSpec((1, N, OUT), lambda p: (p, 0, 0)),
                  pl.BlockSpec((1, NHID, OUT), lambda p: (p, 0, 0)),
                  pl.BlockSpec((1, OUT), lambda p: (0, 0)),
                  pl.BlockSpec((NHID, OUT), lambda p: (0, 0)),
                  pl.BlockSpec((1, OUT), lambda p: (0, 0))],
        out_specs=pl.BlockSpec((N, OUT), lambda p: (0, 0)),
        out_shape=jax.ShapeDtypeStruct((N, OUT), jnp.float32),
        scratch_shapes=[pltpu.VMEM((N, NHID), jnp.float32)],
    )(spec, col1w, col1b, col2w, col2b)


def _tc4(spec, r1, wt, wp, raw1b, raw2w):
    def body(spec_ref, r1_ref, wt_ref, wp_ref, r1b_ref, r2w_ref,
             hsp_ref, u1_ref, v2_ref):
        p = pl.program_id(0)

        @pl.when(p == 0)
        def _():
            hsp_ref[...] = spec_ref[0] * wp_ref[0]
            u1_ref[...] = r1_ref[0] * wt_ref[0]

        @pl.when(p > 0)
        def _():
            hsp_ref[...] = hsp_ref[...] + spec_ref[0] * wp_ref[0]
            u1_ref[...] = u1_ref[...] + r1_ref[0] * wt_ref[0]

        @pl.when(p == P - 1)
        def _():
            u1 = u1_ref[...] + r1b_ref[...]
            u1_ref[...] = u1
            v = jnp.dot(u1, r2w_ref[...], preferred_element_type=jnp.float32)
            v2_ref[0] = v[:, :32]
            v2_ref[1] = v[:, 32:]

    return pl.pallas_call(
        body,
        grid=(P,),
        in_specs=[pl.BlockSpec((1, N, OUT), lambda p: (p, 0, 0)),
                  pl.BlockSpec((1, N, OUT), lambda p: (p, 0, 0)),
                  pl.BlockSpec((1, 1, 1), lambda p: (p, 0, 0)),
                  pl.BlockSpec((1, 1, 1), lambda p: (p, 0, 0)),
                  pl.BlockSpec((1, OUT), lambda p: (0, 0)),
                  pl.BlockSpec((NHID, OUT), lambda p: (0, 0))],
        out_specs=[pl.BlockSpec((N, OUT), lambda p: (0, 0)),
                   pl.BlockSpec((N, OUT), lambda p: (0, 0)),
                   pl.BlockSpec((NC, N, 32), lambda p: (0, 0, 0))],
        out_shape=[jax.ShapeDtypeStruct((N, OUT), jnp.float32),
                   jax.ShapeDtypeStruct((N, OUT), jnp.float32),
                   jax.ShapeDtypeStruct((NC, N, 32), jnp.float32)],
    )(spec, r1, wt, wp, raw1b, raw2w)


def _tc4b(vals3, wt):
    def body(v_ref, wt_ref, o_ref):
        o_ref[...] = v_ref[...] * wt_ref[...]

    return pl.pallas_call(
        body,
        grid=(P,),
        in_specs=[pl.BlockSpec((1, E // 128, 128), lambda p: (p, 0, 0)),
                  pl.BlockSpec((1, 1, 1), lambda p: (p, 0, 0))],
        out_specs=pl.BlockSpec((1, E // 128, 128), lambda p: (p, 0, 0)),
        out_shape=jax.ShapeDtypeStruct((P, E // 128, 128), jnp.float32),
    )(vals3, wt)


def _tc5(hsp, hsh, hcol, u1, s6, raw2b, projw, projb):
    def body(hsp_ref, hsh_ref, hcol_ref, u1_ref, s6_ref, r2b_ref,
             pw_ref, pb_ref, out_ref, hraw_ref):
        u2 = jnp.concatenate([s6_ref[0], s6_ref[1]], axis=1) + r2b_ref[...]
        hraw = (u1_ref[...] + u2) * 0.5
        hraw_ref[...] = hraw
        all_feat = jnp.concatenate(
            [hsp_ref[...], hsh_ref[...], hcol_ref[...], hraw], axis=1)
        out_ref[...] = jnp.dot(all_feat, pw_ref[...],
                               preferred_element_type=jnp.float32) + pb_ref[...]

    return pl.pallas_call(
        body,
        out_shape=[jax.ShapeDtypeStruct((N, OUT), jnp.float32),
                   jax.ShapeDtypeStruct((N, OUT), jnp.float32)],
    )(hsp, hsh, hcol, u1, s6, raw2b, projw, projb)


def kernel(feature, edge_index_0, edge_index_1, edge_index_2,
           vals_0, vals_1, vals_2, params):
    src_all = jnp.concatenate([edge_index_0[1], edge_index_1[1], edge_index_2[1]])
    dst_all = jnp.concatenate([edge_index_0[0], edge_index_1[0], edge_index_2[0]])
    vals_all = jnp.stack([vals_0, vals_1, vals_2])
    vals_flat = vals_all.reshape(P * E)

    w2cat = jnp.stack([
        jnp.concatenate([params["spec1_W_" + str(p)], params["sh1_W"],
                         params["raw1_W"]], axis=1)
        for p in range(P)])
    x2 = _tc1(feature, w2cat)

    s2 = _spmm192(x2.reshape(NC * P * N, 96), src_all, dst_all,
                  vals_flat)[:, :, :N]

    b1s = jnp.stack([params["spec1_b_" + str(p)] for p in range(P)]).reshape(P, 1, NHID)
    w2s = jnp.stack([params["spec2_W_" + str(p)] for p in range(P)])
    x4 = _tc2(s2, b1s, w2s, params["sh1_b"].reshape(1, NHID), params["sh2_W"])

    s4 = _spmm128(x4.reshape(NC * P * N, 64), src_all, dst_all,
                  vals_flat)[:, :, :N]

    b2s = jnp.stack([params["spec2_b_" + str(p)] for p in range(P)]).reshape(P, 1, OUT)
    spec, shm, hsh, ps = _tc3(s4, b2s, params["sh2_b"].reshape(1, OUT))
    hcol = _tc3b(spec, params["col1_W"].reshape(P, NHID, OUT),
                 params["col1_b"].reshape(1, OUT), params["col2_W"],
                 params["col2_b"].reshape(1, OUT))

    ps = ps.reshape(P, 3 * OUT)
    sim = ps @ ps.T / (np.sqrt(3.0 * OUT) * params["tau"])
    t_mat = jax.nn.softmax(sim, axis=1)
    pi0 = jax.nn.softmax(params["weight_b"].squeeze())
    pi = pi0
    for _ in range(13):
        pi = 0.2 * pi0 + 0.8 * (pi @ t_mat)
    wt = pi.reshape(P, 1, 1)
    wp = jax.nn.softmax(pi).reshape(P, 1, 1)

    r1 = s2[1, :, :, 32:96]
    hsp, u1, v2 = _tc4(spec, r1, wt, wp,
                       params["raw1_b"].reshape(1, OUT),
                       params["raw2_W"])
    vals6 = _tc4b(vals_all.reshape(P, E // 128, 128), wt)

    s6 = _spmm64(v2.reshape(NC * N, 32), src_all, dst_all,
                 vals6.reshape(P * E))[:, :N]

    out, hraw = _tc5(hsp, hsh, hcol, u1, s6,
                     params["raw2_b"].reshape(1, OUT), params["proj_W"],
                     params["proj_b"].reshape(1, OUT))

    return (out, spec[0], spec[1], spec[2], shm[0], shm[1], shm[2],
            hcol, hraw)


# speedup vs baseline: 5.1442x; 1.9385x over previous
import functools

import numpy as np
import jax
import jax.numpy as jnp
from jax import lax
from jax.experimental import pallas as pl
from jax.experimental.pallas import tpu as pltpu
from jax.experimental.pallas import tpu_sc as plsc

N = 10000
E = 320000
NFEAT = 128
NHID = 64
OUT = 64
P = 3

NC = 2
NS = 16
NP_ = 10240
RT = NP_ // NS
K = 80
EPT = E // NS
CH = EPT // K


def _telu(x):
    return x * jnp.tanh(jnp.exp(x))


def _make_spmm(Wh, fuse, x_shared):
    WREG = Wh // 16
    mesh = plsc.VectorSubcoreMesh(core_axis_name="c", subcore_axis_name="s")
    out_type = jax.ShapeDtypeStruct(
        (NC, NP_, Wh) if fuse else (NC, P, NP_, Wh), jnp.float32)
    scratch = [
        [pltpu.VMEM((K,), jnp.int32)] * 2,
        [pltpu.VMEM((K,), jnp.int32)] * 2,
        [pltpu.VMEM((K,), jnp.float32)] * 2,
        [pltpu.VMEM((K, Wh), jnp.float32)] * 2,
        pltpu.VMEM((128, Wh), jnp.float32),
        pltpu.VMEM_SHARED((NP_, Wh), jnp.float32),
        [pltpu.SemaphoreType.DMA] * 2,
        [pltpu.SemaphoreType.DMA] * 2,
        [pltpu.SemaphoreType.DMA] * 2,
    ]

    @functools.partial(pl.kernel, out_type=out_type, mesh=mesh,
                       scratch_types=scratch,
                       compiler_params=pltpu.CompilerParams(
                           use_tc_tiling_on_sc=False))
    def k(x_hbm, src_hbm, dst_hbm, vals_hbm, out_hbm,
          srcv, dstv, valv, rows, zbuf, acc, semg, sems, semi):
        c = lax.axis_index("c")
        s = lax.axis_index("s")
        ebase = s * EPT
        rbase = s * RT

        zero16 = jnp.zeros((16,), jnp.float32)

        def zb(i, carry):
            for w in range(WREG):
                zbuf[i, pl.ds(w * 16, 16)] = zero16
            return carry

        lax.fori_loop(0, 128, zb, 0)

        def zacc(t, carry):
            pltpu.sync_copy(zbuf, acc.at[pl.ds(rbase + t * 128, 128)])
            return carry

        lax.fori_loop(0, RT // 128, zacc, 0)
        plsc.subcore_barrier()

        for p in range(P):
            xoff = (c * N) if x_shared else ((c * P + p) * N)

            def chunk_pair(i2, carry):
                descs = []
                for b in range(2):
                    base = p * E + ebase + (2 * i2 + b) * K
                    d1 = pltpu.async_copy(src_hbm.at[pl.ds(base, K)],
                                          srcv[b], semi[b])
                    d2 = pltpu.async_copy(dst_hbm.at[pl.ds(base, K)],
                                          dstv[b], semi[b])
                    d3 = pltpu.async_copy(vals_hbm.at[pl.ds(base, K)],
                                          valv[b], semi[b])
                    descs.append((d1, d2, d3))
                gd = []
                for b in range(2):
                    for d in descs[b]:
                        d.wait()
                    for w in range(K // 16):
                        sl = pl.ds(w * 16, 16)
                        srcv[b][sl] = srcv[b][sl] + xoff
                    gd.append(pltpu.async_copy(x_hbm.at[srcv[b]],
                                               rows[b], semg[b]))
                sd = []
                for b in range(2):
                    gd[b].wait()

                    def scale(g, carry2, b=b):
                        vv = valv[b][pl.ds(g * 16, 16)]
                        for j in range(16):
                            v = vv[j]
                            e = g * 16 + j
                            for w in range(WREG):
                                sl = pl.ds(w * 16, 16)
                                rows[b][e, sl] = rows[b][e, sl] * v
                        return carry2

                    lax.fori_loop(0, K // 16, scale, 0)
                    sd.append(pltpu.async_copy(rows[b], acc.at[dstv[b]],
                                               sems[b], add=True))
                for b in range(2):
                    sd[b].wait()
                return carry

            lax.fori_loop(0, CH // 2, chunk_pair, 0)

            if not fuse:
                plsc.subcore_barrier()
                pltpu.sync_copy(acc.at[pl.ds(rbase, RT)],
                                out_hbm.at[c, p, pl.ds(rbase, RT)])
                if p < P - 1:
                    lax.fori_loop(0, RT // 128, zacc, 0)
                    plsc.subcore_barrier()
        if fuse:
            plsc.subcore_barrier()
            pltpu.sync_copy(acc.at[pl.ds(rbase, RT)],
                            out_hbm.at[c, pl.ds(rbase, RT)])

    return k


_spmm192 = _make_spmm(96, fuse=False, x_shared=False)
_spmm128 = _make_spmm(64, fuse=False, x_shared=False)
_spmm64 = _make_spmm(32, fuse=True, x_shared=True)


def _tc1(feature, w2cat):
    def body(f_ref, w_ref, o_ref):
        res = jnp.dot(f_ref[...], w_ref[0], preferred_element_type=jnp.float32)
        o_ref[0, 0] = res[:, :96]
        o_ref[1, 0] = res[:, 96:]

    return pl.pallas_call(
        body,
        grid=(P,),
        in_specs=[pl.BlockSpec((N, NFEAT), lambda p: (0, 0)),
                  pl.BlockSpec((1, NFEAT, 192), lambda p: (p, 0, 0))],
        out_specs=pl.BlockSpec((NC, 1, N, 96), lambda p: (0, p, 0, 0)),
        out_shape=jax.ShapeDtypeStruct((NC, P, N, 96), jnp.float32),
    )(feature, w2cat)


def _tc2(s2, b1s, w2s, sh1b, sh2w):
    def body(s_ref, b1_ref, w2_ref, shb_ref, shw_ref, o_ref):
        sspec = s_ref[0, 0, :, :64] + b1_ref[0]
        a = jnp.dot(_telu(sspec), w2_ref[0], preferred_element_type=jnp.float32)
        ssh = jnp.concatenate([s_ref[0, 0, :, 64:96], s_ref[1, 0, :, :32]],
                              axis=1) + shb_ref[...]
        b = jnp.dot(_telu(ssh), shw_ref[...], preferred_element_type=jnp.float32)
        o_ref[0, 0] = a
        o_ref[1, 0] = b

    return pl.pallas_call(
        body,
        grid=(P,),
        in_specs=[pl.BlockSpec((NC, 1, N, 96), lambda p: (0, p, 0, 0)),
                  pl.BlockSpec((1, 1, NHID), lambda p: (p, 0, 0)),
                  pl.BlockSpec((1, NHID, OUT), lambda p: (p, 0, 0)),
                  pl.BlockSpec((1, NHID), lambda p: (0, 0)),
                  pl.BlockSpec((NHID, OUT), lambda p: (0, 0))],
        out_specs=pl.BlockSpec((NC, 1, N, OUT), lambda p: (0, p, 0, 0)),
        out_shape=jax.ShapeDtypeStruct((NC, P, N, OUT), jnp.float32),
    )(s2, b1s, w2s, sh1b, sh2w)


def _tc3(s4, b2s, sh2b):
    def body(s_ref, b2_ref, shb_ref, spec_ref, shm_ref, hsh_ref, ps_ref):
        p = pl.program_id(0)
        spec = s_ref[0, 0] + b2_ref[0]
        shm = s_ref[1, 0] + shb_ref[...]
        spec_ref[0] = spec
        shm_ref[0] = shm
        mp = jnp.mean(spec, axis=0)
        mx = jnp.max(spec, axis=0)
        z = jnp.exp(spec - mx[None, :])
        prob = z / jnp.sum(z, axis=0)[None, :]
        ent = -jnp.sum(prob * jnp.log(prob + 1e-06), axis=0)
        ps_ref[0, 0] = jnp.concatenate([mp, mx, ent], axis=-1)

        @pl.when(p == 0)
        def _():
            hsh_ref[...] = shm

        @pl.when(p > 0)
        def _():
            hsh_ref[...] = hsh_ref[...] + shm

        @pl.when(p == P - 1)
        def _():
            hsh_ref[...] = hsh_ref[...] * (1.0 / P)

    return pl.pallas_call(
        body,
        grid=(P,),
        in_specs=[pl.BlockSpec((NC, 1, N, OUT), lambda p: (0, p, 0, 0)),
                  pl.BlockSpec((1, 1, OUT), lambda p: (p, 0, 0)),
                  pl.BlockSpec((1, OUT), lambda p: (0, 0))],
        out_specs=[pl.BlockSpec((1, N, OUT), lambda p: (p, 0, 0)),
                   pl.BlockSpec((1, N, OUT), lambda p: (p, 0, 0)),
                   pl.BlockSpec((N, OUT), lambda p: (0, 0)),
                   pl.BlockSpec((1, 1, 3 * OUT), lambda p: (p, 0, 0))],
        out_shape=[jax.ShapeDtypeStruct((P, N, OUT), jnp.float32),
                   jax.ShapeDtypeStruct((P, N, OUT), jnp.float32),
                   jax.ShapeDtypeStruct((N, OUT), jnp.float32),
                   jax.ShapeDtypeStruct((P, 1, 3 * OUT), jnp.float32)],
    )(s4, b2s, sh2b)


def _tc3b(spec, col1w, col1b, col2w, col2b):
    def body(spec_ref, c1w_ref, c1b_ref, c2w_ref, c2b_ref, hcol_ref, acc_ref):
        p = pl.program_id(0)
        contrib = jnp.dot(spec_ref[0], c1w_ref[0],
                          preferred_element_type=jnp.float32)

        @pl.when(p == 0)
        def _():
            acc_ref[...] = contrib

        @pl.when(p > 0)
        def _():
            acc_ref[...] = acc_ref[...] + contrib

        @pl.when(p == P - 1)
        def _():
            h = jax.nn.relu(acc_ref[...] + c1b_ref[...])
            hcol_ref[...] = jnp.dot(h, c2w_ref[...],
                                    preferred_element_type=jnp.float32) + c2b_ref[...]

    return pl.pallas_call(
        body,
        grid=(P,),
        in_specs=[pl.BlockSpec((1, N, OUT), lambda p: (p, 0, 0)),
                  pl.BlockSpec((1, NHID, OUT), lambda p: (p, 0, 0)),
                  pl.BlockSpec((1, OUT), lambda p: (0, 0)),
                  pl.BlockSpec((NHID, OUT), lambda p: (0, 0)),
                  pl.BlockSpec((1, OUT), lambda p: (0, 0))],
        out_specs=pl.BlockSpec((N, OUT), lambda p: (0, 0)),
        out_shape=jax.ShapeDtypeStruct((N, OUT), jnp.float32),
        scratch_shapes=[pltpu.VMEM((N, NHID), jnp.float32)],
    )(spec, col1w, col1b, col2w, col2b)


def _tc4(spec, r1, wt, wp, raw1b, raw2w):
    def body(spec_ref, r1_ref, wt_ref, wp_ref, r1b_ref, r2w_ref,
             hsp_ref, u1_ref, v2_ref):
        p = pl.program_id(0)

        @pl.when(p == 0)
        def _():
            hsp_ref[...] = spec_ref[0] * wp_ref[0]
            u1_ref[...] = r1_ref[0] * wt_ref[0]

        @pl.when(p > 0)
        def _():
            hsp_ref[...] = hsp_ref[...] + spec_ref[0] * wp_ref[0]
            u1_ref[...] = u1_ref[...] + r1_ref[0] * wt_ref[0]

        @pl.when(p == P - 1)
        def _():
            u1 = u1_ref[...] + r1b_ref[...]
            u1_ref[...] = u1
            v = jnp.dot(u1, r2w_ref[...], preferred_element_type=jnp.float32)
            v2_ref[0] = v[:, :32]
            v2_ref[1] = v[:, 32:]

    return pl.pallas_call(
        body,
        grid=(P,),
        in_specs=[pl.BlockSpec((1, N, OUT), lambda p: (p, 0, 0)),
                  pl.BlockSpec((1, N, OUT), lambda p: (p, 0, 0)),
                  pl.BlockSpec((1, 1, 1), lambda p: (p, 0, 0)),
                  pl.BlockSpec((1, 1, 1), lambda p: (p, 0, 0)),
                  pl.BlockSpec((1, OUT), lambda p: (0, 0)),
                  pl.BlockSpec((NHID, OUT), lambda p: (0, 0))],
        out_specs=[pl.BlockSpec((N, OUT), lambda p: (0, 0)),
                   pl.BlockSpec((N, OUT), lambda p: (0, 0)),
                   pl.BlockSpec((NC, N, 32), lambda p: (0, 0, 0))],
        out_shape=[jax.ShapeDtypeStruct((N, OUT), jnp.float32),
                   jax.ShapeDtypeStruct((N, OUT), jnp.float32),
                   jax.ShapeDtypeStruct((NC, N, 32), jnp.float32)],
    )(spec, r1, wt, wp, raw1b, raw2w)


def _tc4b(vals3, wt):
    def body(v_ref, wt_ref, o_ref):
        o_ref[...] = v_ref[...] * wt_ref[...]

    return pl.pallas_call(
        body,
        grid=(P,),
        in_specs=[pl.BlockSpec((1, E // 128, 128), lambda p: (p, 0, 0)),
                  pl.BlockSpec((1, 1, 1), lambda p: (p, 0, 0))],
        out_specs=pl.BlockSpec((1, E // 128, 128), lambda p: (p, 0, 0)),
        out_shape=jax.ShapeDtypeStruct((P, E // 128, 128), jnp.float32),
    )(vals3, wt)


def _tc5(hsp, hsh, hcol, u1, s6, raw2b, projw, projb):
    def body(hsp_ref, hsh_ref, hcol_ref, u1_ref, s6_ref, r2b_ref,
             pw_ref, pb_ref, out_ref, hraw_ref):
        u2 = jnp.concatenate([s6_ref[0], s6_ref[1]], axis=1) + r2b_ref[...]
        hraw = (u1_ref[...] + u2) * 0.5
        hraw_ref[...] = hraw
        all_feat = jnp.concatenate(
            [hsp_ref[...], hsh_ref[...], hcol_ref[...], hraw], axis=1)
        out_ref[...] = jnp.dot(all_feat, pw_ref[...],
                               preferred_element_type=jnp.float32) + pb_ref[...]

    return pl.pallas_call(
        body,
        out_shape=[jax.ShapeDtypeStruct((N, OUT), jnp.float32),
                   jax.ShapeDtypeStruct((N, OUT), jnp.float32)],
    )(hsp, hsh, hcol, u1, s6, raw2b, projw, projb)


def kernel(feature, edge_index_0, edge_index_1, edge_index_2,
           vals_0, vals_1, vals_2, params):
    src_all = jnp.concatenate([edge_index_0[1], edge_index_1[1], edge_index_2[1]])
    dst_all = jnp.concatenate([edge_index_0[0], edge_index_1[0], edge_index_2[0]])
    vals_all = jnp.stack([vals_0, vals_1, vals_2])
    vals_flat = vals_all.reshape(P * E)

    w2cat = jnp.stack([
        jnp.concatenate([params["spec1_W_" + str(p)], params["sh1_W"],
                         params["raw1_W"]], axis=1)
        for p in range(P)])
    x2 = _tc1(feature, w2cat)

    s2 = _spmm192(x2.reshape(NC * P * N, 96), src_all, dst_all,
                  vals_flat)[:, :, :N]

    b1s = jnp.stack([params["spec1_b_" + str(p)] for p in range(P)]).reshape(P, 1, NHID)
    w2s = jnp.stack([params["spec2_W_" + str(p)] for p in range(P)])
    x4 = _tc2(s2, b1s, w2s, params["sh1_b"].reshape(1, NHID), params["sh2_W"])

    s4 = _spmm128(x4.reshape(NC * P * N, 64), src_all, dst_all,
                  vals_flat)[:, :, :N]

    b2s = jnp.stack([params["spec2_b_" + str(p)] for p in range(P)]).reshape(P, 1, OUT)
    spec, shm, hsh, ps = _tc3(s4, b2s, params["sh2_b"].reshape(1, OUT))
    hcol = _tc3b(spec, params["col1_W"].reshape(P, NHID, OUT),
                 params["col1_b"].reshape(1, OUT), params["col2_W"],
                 params["col2_b"].reshape(1, OUT))

    ps = ps.reshape(P, 3 * OUT)
    sim = ps @ ps.T / (np.sqrt(3.0 * OUT) * params["tau"])
    t_mat = jax.nn.softmax(sim, axis=1)
    pi0 = jax.nn.softmax(params["weight_b"].squeeze())
    pi = pi0
    for _ in range(13):
        pi = 0.2 * pi0 + 0.8 * (pi @ t_mat)
    wt = pi.reshape(P, 1, 1)
    wp = jax.nn.softmax(pi).reshape(P, 1, 1)

    r1 = s2[1, :, :, 32:96]
    hsp, u1, v2 = _tc4(spec, r1, wt, wp,
                       params["raw1_b"].reshape(1, OUT),
                       params["raw2_W"])
    vals6 = _tc4b(vals_all.reshape(P, E // 128, 128), wt)

    s6 = _spmm64(v2.reshape(NC * N, 32), src_all, dst_all,
                 vals6.reshape(P * E))[:, :N]

    out, hraw = _tc5(hsp, hsh, hcol, u1, s6,
                     params["raw2_b"].reshape(1, OUT), params["proj_W"],
                     params["proj_b"].reshape(1, OUT))

    return (out, spec[0], spec[1], spec[2], shm[0], shm[1], shm[2],
            hcol, hraw)


# speedup vs baseline: 6.4225x; 1.2485x over previous
import functools

import numpy as np
import jax
import jax.numpy as jnp
from jax import lax
from jax.experimental import pallas as pl
from jax.experimental.pallas import tpu as pltpu
from jax.experimental.pallas import tpu_sc as plsc

N = 10000
E = 320000
NFEAT = 128
NHID = 64
OUT = 64
P = 3

NC = 2
NS = 16
NP_ = 10240
RT = NP_ // NS
K = 80
EPT = E // NS
CH = EPT // K


def _telu(x):
    return x * jnp.tanh(jnp.exp(x))


def _make_spmm(Wh, fuse, x_shared):
    WREG = Wh // 16
    mesh = plsc.VectorSubcoreMesh(core_axis_name="c", subcore_axis_name="s")
    out_type = jax.ShapeDtypeStruct(
        (NC, NP_, Wh) if fuse else (NC, P, NP_, Wh), jnp.float32)
    scratch = [
        pltpu.VMEM((EPT,), jnp.int32),
        pltpu.VMEM((CH, K), jnp.int32),
        [pltpu.VMEM((K,), jnp.float32)] * 2,
        [pltpu.VMEM((K, Wh), jnp.float32)] * 2,
        pltpu.VMEM((64, Wh), jnp.float32),
        pltpu.VMEM_SHARED((NP_, Wh), jnp.float32),
        [pltpu.SemaphoreType.DMA] * 2,
        [pltpu.SemaphoreType.DMA] * 2,
        [pltpu.SemaphoreType.DMA] * 2,
    ]

    @functools.partial(pl.kernel, out_type=out_type, mesh=mesh,
                       scratch_types=scratch,
                       compiler_params=pltpu.CompilerParams(
                           use_tc_tiling_on_sc=False))
    def k(x_hbm, src_hbm, dst_hbm, vals_hbm, out_hbm,
          srcbuf, dstbuf, valv, rows, zbuf, acc, semg, sems, semi):
        c = lax.axis_index("c")
        s = lax.axis_index("s")
        ebase = s * EPT
        rbase = s * RT

        zero16 = jnp.zeros((16,), jnp.float32)

        def zb(i, carry):
            for w in range(WREG):
                zbuf[i, pl.ds(w * 16, 16)] = zero16
            return carry

        lax.fori_loop(0, 64, zb, 0)

        def zacc(t, carry):
            pltpu.sync_copy(zbuf, acc.at[pl.ds(rbase + t * 64, 64)])
            return carry

        lax.fori_loop(0, RT // 64, zacc, 0)
        plsc.subcore_barrier()

        for p in range(P):
            xoff = (c * N) if x_shared else ((c * P + p) * N)

            pltpu.sync_copy(src_hbm.at[pl.ds(p * E + ebase, EPT)], srcbuf)
            pltpu.sync_copy(dst_hbm.at[pl.ds((p * NS + s) * CH, CH)], dstbuf)

            def addoff(g, carry):
                sl = pl.ds(g * 16, 16)
                srcbuf[sl] = srcbuf[sl] + xoff
                return carry

            lax.fori_loop(0, EPT // 16, addoff, 0)

            def chunk_pair(j, carry):
                i0 = 2 * j
                vd = []
                for b in range(2):
                    base = p * E + ebase + (i0 + b) * K
                    vd.append(pltpu.async_copy(vals_hbm.at[pl.ds(base, K)],
                                               valv[b], semi[b]))
                gd = []
                for b in range(2):
                    @pl.when(j > 0)
                    def _(b=b):
                        pltpu.make_async_copy(rows[b], acc.at[dstbuf.at[0]],
                                              sems[b]).wait()
                    gd.append(pltpu.async_copy(
                        x_hbm.at[srcbuf.at[pl.ds((i0 + b) * K, K)]],
                        rows[b], semg[b]))
                for b in range(2):
                    gd[b].wait()
                    vd[b].wait()

                    def scale(g, carry2, b=b):
                        vv = valv[b][pl.ds(g * 16, 16)]
                        for j2 in range(16):
                            v = vv[j2]
                            e = g * 16 + j2
                            for w in range(WREG):
                                sl = pl.ds(w * 16, 16)
                                rows[b][e, sl] = rows[b][e, sl] * v
                        return carry2

                    lax.fori_loop(0, K // 16, scale, 0)
                    pltpu.async_copy(rows[b], acc.at[dstbuf.at[i0 + b]],
                                     sems[b], add=True)
                return carry

            lax.fori_loop(0, CH // 2, chunk_pair, 0)
            for b in range(2):
                pltpu.make_async_copy(rows[b], acc.at[dstbuf.at[0]],
                                      sems[b]).wait()

            if not fuse:
                plsc.subcore_barrier()
                pltpu.sync_copy(acc.at[pl.ds(rbase, RT)],
                                out_hbm.at[c, p, pl.ds(rbase, RT)])
                if p < P - 1:
                    lax.fori_loop(0, RT // 64, zacc, 0)
                    plsc.subcore_barrier()
        if fuse:
            plsc.subcore_barrier()
            pltpu.sync_copy(acc.at[pl.ds(rbase, RT)],
                            out_hbm.at[c, pl.ds(rbase, RT)])

    return k


_spmm192 = _make_spmm(96, fuse=False, x_shared=False)
_spmm128 = _make_spmm(64, fuse=False, x_shared=False)
_spmm64 = _make_spmm(32, fuse=True, x_shared=True)


def _tc1(feature, w2cat):
    def body(f_ref, w_ref, o_ref):
        res = jnp.dot(f_ref[...], w_ref[0], preferred_element_type=jnp.float32)
        o_ref[0, 0] = res[:, :96]
        o_ref[1, 0] = res[:, 96:]

    return pl.pallas_call(
        body,
        grid=(P,),
        in_specs=[pl.BlockSpec((N, NFEAT), lambda p: (0, 0)),
                  pl.BlockSpec((1, NFEAT, 192), lambda p: (p, 0, 0))],
        out_specs=pl.BlockSpec((NC, 1, N, 96), lambda p: (0, p, 0, 0)),
        out_shape=jax.ShapeDtypeStruct((NC, P, N, 96), jnp.float32),
    )(feature, w2cat)


def _tc2(s2, b1s, w2s, sh1b, sh2w):
    def body(s_ref, b1_ref, w2_ref, shb_ref, shw_ref, o_ref):
        sspec = s_ref[0, 0, :, :64] + b1_ref[0]
        a = jnp.dot(_telu(sspec), w2_ref[0], preferred_element_type=jnp.float32)
        ssh = jnp.concatenate([s_ref[0, 0, :, 64:96], s_ref[1, 0, :, :32]],
                              axis=1) + shb_ref[...]
        b = jnp.dot(_telu(ssh), shw_ref[...], preferred_element_type=jnp.float32)
        o_ref[0, 0] = a
        o_ref[1, 0] = b

    return pl.pallas_call(
        body,
        grid=(P,),
        in_specs=[pl.BlockSpec((NC, 1, N, 96), lambda p: (0, p, 0, 0)),
                  pl.BlockSpec((1, 1, NHID), lambda p: (p, 0, 0)),
                  pl.BlockSpec((1, NHID, OUT), lambda p: (p, 0, 0)),
                  pl.BlockSpec((1, NHID), lambda p: (0, 0)),
                  pl.BlockSpec((NHID, OUT), lambda p: (0, 0))],
        out_specs=pl.BlockSpec((NC, 1, N, OUT), lambda p: (0, p, 0, 0)),
        out_shape=jax.ShapeDtypeStruct((NC, P, N, OUT), jnp.float32),
    )(s2, b1s, w2s, sh1b, sh2w)


def _tc3(s4, b2s, sh2b):
    def body(s_ref, b2_ref, shb_ref, spec_ref, shm_ref, hsh_ref, ps_ref):
        p = pl.program_id(0)
        spec = s_ref[0, 0] + b2_ref[0]
        shm = s_ref[1, 0] + shb_ref[...]
        spec_ref[0] = spec
        shm_ref[0] = shm
        mp = jnp.mean(spec, axis=0)
        mx = jnp.max(spec, axis=0)
        z = jnp.exp(spec - mx[None, :])
        prob = z / jnp.sum(z, axis=0)[None, :]
        ent = -jnp.sum(prob * jnp.log(prob + 1e-06), axis=0)
        ps_ref[0, 0] = jnp.concatenate([mp, mx, ent], axis=-1)

        @pl.when(p == 0)
        def _():
            hsh_ref[...] = shm

        @pl.when(p > 0)
        def _():
            hsh_ref[...] = hsh_ref[...] + shm

        @pl.when(p == P - 1)
        def _():
            hsh_ref[...] = hsh_ref[...] * (1.0 / P)

    return pl.pallas_call(
        body,
        grid=(P,),
        in_specs=[pl.BlockSpec((NC, 1, N, OUT), lambda p: (0, p, 0, 0)),
                  pl.BlockSpec((1, 1, OUT), lambda p: (p, 0, 0)),
                  pl.BlockSpec((1, OUT), lambda p: (0, 0))],
        out_specs=[pl.BlockSpec((1, N, OUT), lambda p: (p, 0, 0)),
                   pl.BlockSpec((1, N, OUT), lambda p: (p, 0, 0)),
                   pl.BlockSpec((N, OUT), lambda p: (0, 0)),
                   pl.BlockSpec((1, 1, 3 * OUT), lambda p: (p, 0, 0))],
        out_shape=[jax.ShapeDtypeStruct((P, N, OUT), jnp.float32),
                   jax.ShapeDtypeStruct((P, N, OUT), jnp.float32),
                   jax.ShapeDtypeStruct((N, OUT), jnp.float32),
                   jax.ShapeDtypeStruct((P, 1, 3 * OUT), jnp.float32)],
    )(s4, b2s, sh2b)


def _tc3b(spec, col1w, col1b, col2w, col2b):
    def body(spec_ref, c1w_ref, c1b_ref, c2w_ref, c2b_ref, hcol_ref, acc_ref):
        p = pl.program_id(0)
        contrib = jnp.dot(spec_ref[0], c1w_ref[0],
                          preferred_element_type=jnp.float32)

        @pl.when(p == 0)
        def _():
            acc_ref[...] = contrib

        @pl.when(p > 0)
        def _():
            acc_ref[...] = acc_ref[...] + contrib

        @pl.when(p == P - 1)
        def _():
            h = jax.nn.relu(acc_ref[...] + c1b_ref[...])
            hcol_ref[...] = jnp.dot(h, c2w_ref[...],
                                    preferred_element_type=jnp.float32) + c2b_ref[...]

    return pl.pallas_call(
        body,
        grid=(P,),
        in_specs=[pl.BlockSpec((1, N, OUT), lambda p: (p, 0, 0)),
                  pl.BlockSpec((1, NHID, OUT), lambda p: (p, 0, 0)),
                  pl.BlockSpec((1, OUT), lambda p: (0, 0)),
                  pl.BlockSpec((NHID, OUT), lambda p: (0, 0)),
                  pl.BlockSpec((1, OUT), lambda p: (0, 0))],
        out_specs=pl.BlockSpec((N, OUT), lambda p: (0, 0)),
        out_shape=jax.ShapeDtypeStruct((N, OUT), jnp.float32),
        scratch_shapes=[pltpu.VMEM((N, NHID), jnp.float32)],
    )(spec, col1w, col1b, col2w, col2b)


def _tc4(spec, r1, wt, wp, raw1b, raw2w):
    def body(spec_ref, r1_ref, wt_ref, wp_ref, r1b_ref, r2w_ref,
             hsp_ref, u1_ref, v2_ref):
        p = pl.program_id(0)

        @pl.when(p == 0)
        def _():
            hsp_ref[...] = spec_ref[0] * wp_ref[0]
            u1_ref[...] = r1_ref[0] * wt_ref[0]

        @pl.when(p > 0)
        def _():
            hsp_ref[...] = hsp_ref[...] + spec_ref[0] * wp_ref[0]
            u1_ref[...] = u1_ref[...] + r1_ref[0] * wt_ref[0]

        @pl.when(p == P - 1)
        def _():
            u1 = u1_ref[...] + r1b_ref[...]
            u1_ref[...] = u1
            v = jnp.dot(u1, r2w_ref[...], preferred_element_type=jnp.float32)
            v2_ref[0] = v[:, :32]
            v2_ref[1] = v[:, 32:]

    return pl.pallas_call(
        body,
        grid=(P,),
        in_specs=[pl.BlockSpec((1, N, OUT), lambda p: (p, 0, 0)),
                  pl.BlockSpec((1, N, OUT), lambda p: (p, 0, 0)),
                  pl.BlockSpec((1, 1, 1), lambda p: (p, 0, 0)),
                  pl.BlockSpec((1, 1, 1), lambda p: (p, 0, 0)),
                  pl.BlockSpec((1, OUT), lambda p: (0, 0)),
                  pl.BlockSpec((NHID, OUT), lambda p: (0, 0))],
        out_specs=[pl.BlockSpec((N, OUT), lambda p: (0, 0)),
                   pl.BlockSpec((N, OUT), lambda p: (0, 0)),
                   pl.BlockSpec((NC, N, 32), lambda p: (0, 0, 0))],
        out_shape=[jax.ShapeDtypeStruct((N, OUT), jnp.float32),
                   jax.ShapeDtypeStruct((N, OUT), jnp.float32),
                   jax.ShapeDtypeStruct((NC, N, 32), jnp.float32)],
    )(spec, r1, wt, wp, raw1b, raw2w)


def _tc4b(vals3, wt):
    def body(v_ref, wt_ref, o_ref):
        o_ref[...] = v_ref[...] * wt_ref[...]

    return pl.pallas_call(
        body,
        grid=(P,),
        in_specs=[pl.BlockSpec((1, E // 128, 128), lambda p: (p, 0, 0)),
                  pl.BlockSpec((1, 1, 1), lambda p: (p, 0, 0))],
        out_specs=pl.BlockSpec((1, E // 128, 128), lambda p: (p, 0, 0)),
        out_shape=jax.ShapeDtypeStruct((P, E // 128, 128), jnp.float32),
    )(vals3, wt)


def _tc5(hsp, hsh, hcol, u1, s6, raw2b, projw, projb):
    def body(hsp_ref, hsh_ref, hcol_ref, u1_ref, s6_ref, r2b_ref,
             pw_ref, pb_ref, out_ref, hraw_ref):
        u2 = jnp.concatenate([s6_ref[0], s6_ref[1]], axis=1) + r2b_ref[...]
        hraw = (u1_ref[...] + u2) * 0.5
        hraw_ref[...] = hraw
        all_feat = jnp.concatenate(
            [hsp_ref[...], hsh_ref[...], hcol_ref[...], hraw], axis=1)
        out_ref[...] = jnp.dot(all_feat, pw_ref[...],
                               preferred_element_type=jnp.float32) + pb_ref[...]

    return pl.pallas_call(
        body,
        out_shape=[jax.ShapeDtypeStruct((N, OUT), jnp.float32),
                   jax.ShapeDtypeStruct((N, OUT), jnp.float32)],
    )(hsp, hsh, hcol, u1, s6, raw2b, projw, projb)


def kernel(feature, edge_index_0, edge_index_1, edge_index_2,
           vals_0, vals_1, vals_2, params):
    src_all = jnp.concatenate([edge_index_0[1], edge_index_1[1], edge_index_2[1]])
    dst_all = jnp.concatenate(
        [edge_index_0[0], edge_index_1[0], edge_index_2[0]]).reshape(
            P * NS * CH, K)
    vals_all = jnp.stack([vals_0, vals_1, vals_2])
    vals_flat = vals_all.reshape(P * E)

    w2cat = jnp.stack([
        jnp.concatenate([params["spec1_W_" + str(p)], params["sh1_W"],
                         params["raw1_W"]], axis=1)
        for p in range(P)])
    x2 = _tc1(feature, w2cat)

    s2 = _spmm192(x2.reshape(NC * P * N, 96), src_all, dst_all,
                  vals_flat)[:, :, :N]

    b1s = jnp.stack([params["spec1_b_" + str(p)] for p in range(P)]).reshape(P, 1, NHID)
    w2s = jnp.stack([params["spec2_W_" + str(p)] for p in range(P)])
    x4 = _tc2(s2, b1s, w2s, params["sh1_b"].reshape(1, NHID), params["sh2_W"])

    s4 = _spmm128(x4.reshape(NC * P * N, 64), src_all, dst_all,
                  vals_flat)[:, :, :N]

    b2s = jnp.stack([params["spec2_b_" + str(p)] for p in range(P)]).reshape(P, 1, OUT)
    spec, shm, hsh, ps = _tc3(s4, b2s, params["sh2_b"].reshape(1, OUT))
    hcol = _tc3b(spec, params["col1_W"].reshape(P, NHID, OUT),
                 params["col1_b"].reshape(1, OUT), params["col2_W"],
                 params["col2_b"].reshape(1, OUT))

    ps = ps.reshape(P, 3 * OUT)
    sim = ps @ ps.T / (np.sqrt(3.0 * OUT) * params["tau"])
    t_mat = jax.nn.softmax(sim, axis=1)
    pi0 = jax.nn.softmax(params["weight_b"].squeeze())
    pi = pi0
    for _ in range(13):
        pi = 0.2 * pi0 + 0.8 * (pi @ t_mat)
    wt = pi.reshape(P, 1, 1)
    wp = jax.nn.softmax(pi).reshape(P, 1, 1)

    r1 = s2[1, :, :, 32:96]
    hsp, u1, v2 = _tc4(spec, r1, wt, wp,
                       params["raw1_b"].reshape(1, OUT),
                       params["raw2_W"])
    vals6 = _tc4b(vals_all.reshape(P, E // 128, 128), wt)

    s6 = _spmm64(v2.reshape(NC * N, 32), src_all, dst_all,
                 vals6.reshape(P * E))[:, :N]

    out, hraw = _tc5(hsp, hsh, hcol, u1, s6,
                     params["raw2_b"].reshape(1, OUT), params["proj_W"],
                     params["proj_b"].reshape(1, OUT))

    return (out, spec[0], spec[1], spec[2], shm[0], shm[1], shm[2],
            hcol, hraw)


# speedup vs baseline: 6.7412x; 1.0496x over previous
import functools

import numpy as np
import jax
import jax.numpy as jnp
from jax import lax
from jax.experimental import pallas as pl
from jax.experimental.pallas import tpu as pltpu
from jax.experimental.pallas import tpu_sc as plsc

N = 10000
E = 320000
NFEAT = 128
NHID = 64
OUT = 64
P = 3

NC = 2
NS = 16
NP_ = 10240
RT = NP_ // NS
K = 80
EPT = E // NS
CH = EPT // K


def _telu(x):
    return x * jnp.tanh(jnp.exp(x))


def _make_spmm(Wh, fuse, x_shared):
    WREG = Wh // 16
    mesh = plsc.VectorSubcoreMesh(core_axis_name="c", subcore_axis_name="s")
    out_type = jax.ShapeDtypeStruct(
        (NC, NP_, Wh) if fuse else (NC, P, NP_, Wh), jnp.float32)
    NB = 4
    scratch = [
        pltpu.VMEM((EPT,), jnp.int32),
        [pltpu.VMEM((K,), jnp.int32)] * 4,
        [pltpu.VMEM((K,), jnp.float32)] * 4,
        [pltpu.VMEM((K, Wh), jnp.float32)] * 4,
        pltpu.VMEM((64, Wh), jnp.float32),
        pltpu.VMEM_SHARED((NP_, Wh), jnp.float32),
        [pltpu.SemaphoreType.DMA] * 4,
        [pltpu.SemaphoreType.DMA] * 4,
        [pltpu.SemaphoreType.DMA] * 4,
    ]

    @functools.partial(pl.kernel, out_type=out_type, mesh=mesh,
                       scratch_types=scratch,
                       compiler_params=pltpu.CompilerParams(
                           use_tc_tiling_on_sc=False))
    def k(x_hbm, src_hbm, dst_hbm, vals_hbm, out_hbm,
          srcbuf, dstv, valv, rows, zbuf, acc, semg, sems, semi):
        c = lax.axis_index("c")
        s = lax.axis_index("s")
        ebase = s * EPT
        rbase = s * RT

        zero16 = jnp.zeros((16,), jnp.float32)

        def zb(i, carry):
            for w in range(WREG):
                zbuf[i, pl.ds(w * 16, 16)] = zero16
            return carry

        lax.fori_loop(0, 64, zb, 0)

        def zacc(t, carry):
            pltpu.sync_copy(zbuf, acc.at[pl.ds(rbase + t * 64, 64)])
            return carry

        lax.fori_loop(0, RT // 64, zacc, 0)
        plsc.subcore_barrier()

        for p in range(P):
            xoff = (c * N) if x_shared else ((c * P + p) * N)

            pltpu.sync_copy(src_hbm.at[pl.ds(p * E + ebase, EPT)], srcbuf)

            def addoff(g, carry):
                sl = pl.ds(g * 16, 16)
                srcbuf[sl] = srcbuf[sl] + xoff
                return carry

            lax.fori_loop(0, EPT // 16, addoff, 0)

            def do_chunks(i0, nb, first):
                for b in range(nb):
                    @pl.when(jnp.logical_not(first))
                    def _(b=b):
                        pltpu.make_async_copy(rows[b], acc.at[dstv[b]],
                                              sems[b]).wait()
                    base = p * E + ebase + (i0 + b) * K
                    pltpu.async_copy(dst_hbm.at[pl.ds(base, K)],
                                     dstv[b], semi[b])
                    pltpu.async_copy(vals_hbm.at[pl.ds(base, K)],
                                     valv[b], semi[b])
                gd = []
                for b in range(nb):
                    gd.append(pltpu.async_copy(
                        x_hbm.at[srcbuf.at[pl.ds((i0 + b) * K, K)]],
                        rows[b], semg[b]))
                for b in range(nb):
                    gd[b].wait()
                    pltpu.make_async_copy(dst_hbm.at[pl.ds(0, K)], dstv[b],
                                          semi[b]).wait()
                    pltpu.make_async_copy(vals_hbm.at[pl.ds(0, K)], valv[b],
                                          semi[b]).wait()

                    def scale(g, carry2, b=b):
                        vv = valv[b][pl.ds(g * 16, 16)]
                        for j2 in range(16):
                            v = vv[j2]
                            e = g * 16 + j2
                            for w in range(WREG):
                                sl = pl.ds(w * 16, 16)
                                rows[b][e, sl] = rows[b][e, sl] * v
                        return carry2

                    lax.fori_loop(0, K // 16, scale, 0)
                    pltpu.async_copy(rows[b], acc.at[dstv[b]],
                                     sems[b], add=True)

            def quad(q, carry):
                do_chunks(4 * q, 4, q == 0)
                return carry

            lax.fori_loop(0, CH // 4, quad, 0)
            if CH % 4:
                do_chunks((CH // 4) * 4, CH % 4, jnp.bool_(False))
            for b in range(4 if CH >= 4 else CH):
                pltpu.make_async_copy(rows[b], acc.at[dstv[b]],
                                      sems[b]).wait()

            if not fuse:
                plsc.subcore_barrier()
                pltpu.sync_copy(acc.at[pl.ds(rbase, RT)],
                                out_hbm.at[c, p, pl.ds(rbase, RT)])
                if p < P - 1:
                    lax.fori_loop(0, RT // 64, zacc, 0)
                    plsc.subcore_barrier()
        if fuse:
            plsc.subcore_barrier()
            pltpu.sync_copy(acc.at[pl.ds(rbase, RT)],
                            out_hbm.at[c, pl.ds(rbase, RT)])

    return k


_spmm192 = _make_spmm(96, fuse=False, x_shared=False)
_spmm128 = _make_spmm(64, fuse=False, x_shared=False)
_spmm64 = _make_spmm(32, fuse=True, x_shared=True)


def _tc1(feature, w2cat):
    def body(f_ref, w_ref, o_ref):
        res = jnp.dot(f_ref[...], w_ref[0], preferred_element_type=jnp.float32)
        o_ref[0, 0] = res[:, :96]
        o_ref[1, 0] = res[:, 96:]

    return pl.pallas_call(
        body,
        grid=(P,),
        in_specs=[pl.BlockSpec((N, NFEAT), lambda p: (0, 0)),
                  pl.BlockSpec((1, NFEAT, 192), lambda p: (p, 0, 0))],
        out_specs=pl.BlockSpec((NC, 1, N, 96), lambda p: (0, p, 0, 0)),
        out_shape=jax.ShapeDtypeStruct((NC, P, N, 96), jnp.float32),
    )(feature, w2cat)


def _tc2(s2, b1s, w2s, sh1b, sh2w):
    def body(s_ref, b1_ref, w2_ref, shb_ref, shw_ref, o_ref):
        sspec = s_ref[0, 0, :, :64] + b1_ref[0]
        a = jnp.dot(_telu(sspec), w2_ref[0], preferred_element_type=jnp.float32)
        ssh = jnp.concatenate([s_ref[0, 0, :, 64:96], s_ref[1, 0, :, :32]],
                              axis=1) + shb_ref[...]
        b = jnp.dot(_telu(ssh), shw_ref[...], preferred_element_type=jnp.float32)
        o_ref[0, 0] = a
        o_ref[1, 0] = b

    return pl.pallas_call(
        body,
        grid=(P,),
        in_specs=[pl.BlockSpec((NC, 1, N, 96), lambda p: (0, p, 0, 0)),
                  pl.BlockSpec((1, 1, NHID), lambda p: (p, 0, 0)),
                  pl.BlockSpec((1, NHID, OUT), lambda p: (p, 0, 0)),
                  pl.BlockSpec((1, NHID), lambda p: (0, 0)),
                  pl.BlockSpec((NHID, OUT), lambda p: (0, 0))],
        out_specs=pl.BlockSpec((NC, 1, N, OUT), lambda p: (0, p, 0, 0)),
        out_shape=jax.ShapeDtypeStruct((NC, P, N, OUT), jnp.float32),
    )(s2, b1s, w2s, sh1b, sh2w)


def _tc3(s4, b2s, sh2b):
    def body(s_ref, b2_ref, shb_ref, spec_ref, shm_ref, hsh_ref, ps_ref):
        p = pl.program_id(0)
        spec = s_ref[0, 0] + b2_ref[0]
        shm = s_ref[1, 0] + shb_ref[...]
        spec_ref[0] = spec
        shm_ref[0] = shm
        mp = jnp.mean(spec, axis=0)
        mx = jnp.max(spec, axis=0)
        z = jnp.exp(spec - mx[None, :])
        prob = z / jnp.sum(z, axis=0)[None, :]
        ent = -jnp.sum(prob * jnp.log(prob + 1e-06), axis=0)
        ps_ref[0, 0] = jnp.concatenate([mp, mx, ent], axis=-1)

        @pl.when(p == 0)
        def _():
            hsh_ref[...] = shm

        @pl.when(p > 0)
        def _():
            hsh_ref[...] = hsh_ref[...] + shm

        @pl.when(p == P - 1)
        def _():
            hsh_ref[...] = hsh_ref[...] * (1.0 / P)

    return pl.pallas_call(
        body,
        grid=(P,),
        in_specs=[pl.BlockSpec((NC, 1, N, OUT), lambda p: (0, p, 0, 0)),
                  pl.BlockSpec((1, 1, OUT), lambda p: (p, 0, 0)),
                  pl.BlockSpec((1, OUT), lambda p: (0, 0))],
        out_specs=[pl.BlockSpec((1, N, OUT), lambda p: (p, 0, 0)),
                   pl.BlockSpec((1, N, OUT), lambda p: (p, 0, 0)),
                   pl.BlockSpec((N, OUT), lambda p: (0, 0)),
                   pl.BlockSpec((1, 1, 3 * OUT), lambda p: (p, 0, 0))],
        out_shape=[jax.ShapeDtypeStruct((P, N, OUT), jnp.float32),
                   jax.ShapeDtypeStruct((P, N, OUT), jnp.float32),
                   jax.ShapeDtypeStruct((N, OUT), jnp.float32),
                   jax.ShapeDtypeStruct((P, 1, 3 * OUT), jnp.float32)],
    )(s4, b2s, sh2b)


def _tc3b(spec, col1w, col1b, col2w, col2b):
    def body(spec_ref, c1w_ref, c1b_ref, c2w_ref, c2b_ref, hcol_ref, acc_ref):
        p = pl.program_id(0)
        contrib = jnp.dot(spec_ref[0], c1w_ref[0],
                          preferred_element_type=jnp.float32)

        @pl.when(p == 0)
        def _():
            acc_ref[...] = contrib

        @pl.when(p > 0)
        def _():
            acc_ref[...] = acc_ref[...] + contrib

        @pl.when(p == P - 1)
        def _():
            h = jax.nn.relu(acc_ref[...] + c1b_ref[...])
            hcol_ref[...] = jnp.dot(h, c2w_ref[...],
                                    preferred_element_type=jnp.float32) + c2b_ref[...]

    return pl.pallas_call(
        body,
        grid=(P,),
        in_specs=[pl.BlockSpec((1, N, OUT), lambda p: (p, 0, 0)),
                  pl.BlockSpec((1, NHID, OUT), lambda p: (p, 0, 0)),
                  pl.BlockSpec((1, OUT), lambda p: (0, 0)),
                  pl.BlockSpec((NHID, OUT), lambda p: (0, 0)),
                  pl.BlockSpec((1, OUT), lambda p: (0, 0))],
        out_specs=pl.BlockSpec((N, OUT), lambda p: (0, 0)),
        out_shape=jax.ShapeDtypeStruct((N, OUT), jnp.float32),
        scratch_shapes=[pltpu.VMEM((N, NHID), jnp.float32)],
    )(spec, col1w, col1b, col2w, col2b)


def _tc4(spec, r1, wt, wp, raw1b, raw2w):
    def body(spec_ref, r1_ref, wt_ref, wp_ref, r1b_ref, r2w_ref,
             hsp_ref, u1_ref, v2_ref):
        p = pl.program_id(0)

        @pl.when(p == 0)
        def _():
            hsp_ref[...] = spec_ref[0] * wp_ref[0]
            u1_ref[...] = r1_ref[0] * wt_ref[0]

        @pl.when(p > 0)
        def _():
            hsp_ref[...] = hsp_ref[...] + spec_ref[0] * wp_ref[0]
            u1_ref[...] = u1_ref[...] + r1_ref[0] * wt_ref[0]

        @pl.when(p == P - 1)
        def _():
            u1 = u1_ref[...] + r1b_ref[...]
            u1_ref[...] = u1
            v = jnp.dot(u1, r2w_ref[...], preferred_element_type=jnp.float32)
            v2_ref[0] = v[:, :32]
            v2_ref[1] = v[:, 32:]

    return pl.pallas_call(
        body,
        grid=(P,),
        in_specs=[pl.BlockSpec((1, N, OUT), lambda p: (p, 0, 0)),
                  pl.BlockSpec((1, N, OUT), lambda p: (p, 0, 0)),
                  pl.BlockSpec((1, 1, 1), lambda p: (p, 0, 0)),
                  pl.BlockSpec((1, 1, 1), lambda p: (p, 0, 0)),
                  pl.BlockSpec((1, OUT), lambda p: (0, 0)),
                  pl.BlockSpec((NHID, OUT), lambda p: (0, 0))],
        out_specs=[pl.BlockSpec((N, OUT), lambda p: (0, 0)),
                   pl.BlockSpec((N, OUT), lambda p: (0, 0)),
                   pl.BlockSpec((NC, N, 32), lambda p: (0, 0, 0))],
        out_shape=[jax.ShapeDtypeStruct((N, OUT), jnp.float32),
                   jax.ShapeDtypeStruct((N, OUT), jnp.float32),
                   jax.ShapeDtypeStruct((NC, N, 32), jnp.float32)],
    )(spec, r1, wt, wp, raw1b, raw2w)


def _tc4b(vals3, wt):
    def body(v_ref, wt_ref, o_ref):
        o_ref[...] = v_ref[...] * wt_ref[...]

    return pl.pallas_call(
        body,
        grid=(P,),
        in_specs=[pl.BlockSpec((1, E // 128, 128), lambda p: (p, 0, 0)),
                  pl.BlockSpec((1, 1, 1), lambda p: (p, 0, 0))],
        out_specs=pl.BlockSpec((1, E // 128, 128), lambda p: (p, 0, 0)),
        out_shape=jax.ShapeDtypeStruct((P, E // 128, 128), jnp.float32),
    )(vals3, wt)


def _tc5(hsp, hsh, hcol, u1, s6, raw2b, projw, projb):
    def body(hsp_ref, hsh_ref, hcol_ref, u1_ref, s6_ref, r2b_ref,
             pw_ref, pb_ref, out_ref, hraw_ref):
        u2 = jnp.concatenate([s6_ref[0], s6_ref[1]], axis=1) + r2b_ref[...]
        hraw = (u1_ref[...] + u2) * 0.5
        hraw_ref[...] = hraw
        all_feat = jnp.concatenate(
            [hsp_ref[...], hsh_ref[...], hcol_ref[...], hraw], axis=1)
        out_ref[...] = jnp.dot(all_feat, pw_ref[...],
                               preferred_element_type=jnp.float32) + pb_ref[...]

    return pl.pallas_call(
        body,
        out_shape=[jax.ShapeDtypeStruct((N, OUT), jnp.float32),
                   jax.ShapeDtypeStruct((N, OUT), jnp.float32)],
    )(hsp, hsh, hcol, u1, s6, raw2b, projw, projb)


def kernel(feature, edge_index_0, edge_index_1, edge_index_2,
           vals_0, vals_1, vals_2, params):
    src_all = jnp.concatenate([edge_index_0[1], edge_index_1[1], edge_index_2[1]])
    dst_all = jnp.concatenate(
        [edge_index_0[0], edge_index_1[0], edge_index_2[0]])
    vals_all = jnp.stack([vals_0, vals_1, vals_2])
    vals_flat = vals_all.reshape(P * E)

    w2cat = jnp.stack([
        jnp.concatenate([params["spec1_W_" + str(p)], params["sh1_W"],
                         params["raw1_W"]], axis=1)
        for p in range(P)])
    x2 = _tc1(feature, w2cat)

    s2 = _spmm192(x2.reshape(NC * P * N, 96), src_all, dst_all,
                  vals_flat)[:, :, :N]

    b1s = jnp.stack([params["spec1_b_" + str(p)] for p in range(P)]).reshape(P, 1, NHID)
    w2s = jnp.stack([params["spec2_W_" + str(p)] for p in range(P)])
    x4 = _tc2(s2, b1s, w2s, params["sh1_b"].reshape(1, NHID), params["sh2_W"])

    s4 = _spmm128(x4.reshape(NC * P * N, 64), src_all, dst_all,
                  vals_flat)[:, :, :N]

    b2s = jnp.stack([params["spec2_b_" + str(p)] for p in range(P)]).reshape(P, 1, OUT)
    spec, shm, hsh, ps = _tc3(s4, b2s, params["sh2_b"].reshape(1, OUT))
    hcol = _tc3b(spec, params["col1_W"].reshape(P, NHID, OUT),
                 params["col1_b"].reshape(1, OUT), params["col2_W"],
                 params["col2_b"].reshape(1, OUT))

    ps = ps.reshape(P, 3 * OUT)
    sim = ps @ ps.T / (np.sqrt(3.0 * OUT) * params["tau"])
    t_mat = jax.nn.softmax(sim, axis=1)
    pi0 = jax.nn.softmax(params["weight_b"].squeeze())
    pi = pi0
    for _ in range(13):
        pi = 0.2 * pi0 + 0.8 * (pi @ t_mat)
    wt = pi.reshape(P, 1, 1)
    wp = jax.nn.softmax(pi).reshape(P, 1, 1)

    r1 = s2[1, :, :, 32:96]
    hsp, u1, v2 = _tc4(spec, r1, wt, wp,
                       params["raw1_b"].reshape(1, OUT),
                       params["raw2_W"])
    vals6 = _tc4b(vals_all.reshape(P, E // 128, 128), wt)

    s6 = _spmm64(v2.reshape(NC * N, 32), src_all, dst_all,
                 vals6.reshape(P * E))[:, :N]

    out, hraw = _tc5(hsp, hsh, hcol, u1, s6,
                     params["raw2_b"].reshape(1, OUT), params["proj_W"],
                     params["proj_b"].reshape(1, OUT))

    return (out, spec[0], spec[1], spec[2], shm[0], shm[1], shm[2],
            hcol, hraw)


# speedup vs baseline: 6.8677x; 1.0188x over previous
import functools

import numpy as np
import jax
import jax.numpy as jnp
from jax import lax
from jax.experimental import pallas as pl
from jax.experimental.pallas import tpu as pltpu
from jax.experimental.pallas import tpu_sc as plsc

N = 10000
E = 320000
NFEAT = 128
NHID = 64
OUT = 64
P = 3

NC = 2
NS = 16
NP_ = 10240
RT = NP_ // NS
EPT = E // NS


def _telu(x):
    return x * jnp.tanh(jnp.exp(x))


def _make_spmm(Wh, fuse, x_shared, K=128, NB=4):
    WREG = Wh // 16
    CH = EPT // K
    TK = EPT - CH * K
    mesh = plsc.VectorSubcoreMesh(core_axis_name="c", subcore_axis_name="s")
    out_type = jax.ShapeDtypeStruct(
        (NC, NP_, Wh) if fuse else (NC, P, NP_, Wh), jnp.float32)
    scratch = [
        pltpu.VMEM((EPT,), jnp.int32),
        [pltpu.VMEM((K,), jnp.int32)] * NB,
        [pltpu.VMEM((K,), jnp.float32)] * NB,
        [pltpu.VMEM((K, Wh), jnp.float32)] * NB,
        pltpu.VMEM((TK,), jnp.int32),
        pltpu.VMEM((TK,), jnp.float32),
        pltpu.VMEM((TK, Wh), jnp.float32),
        pltpu.VMEM((64, Wh), jnp.float32),
        pltpu.VMEM_SHARED((NP_, Wh), jnp.float32),
        [pltpu.SemaphoreType.DMA] * NB,
        [pltpu.SemaphoreType.DMA] * NB,
        [pltpu.SemaphoreType.DMA] * NB,
    ]

    @functools.partial(pl.kernel, out_type=out_type, mesh=mesh,
                       scratch_types=scratch,
                       compiler_params=pltpu.CompilerParams(
                           use_tc_tiling_on_sc=False))
    def k(x_hbm, src_hbm, dst_hbm, vals_hbm, out_hbm,
          srcbuf, dstv, valv, rows, tdst, tval, trows, zbuf, acc,
          semg, sems, semi):
        c = lax.axis_index("c")
        s = lax.axis_index("s")
        ebase = s * EPT
        rbase = s * RT

        zero16 = jnp.zeros((16,), jnp.float32)

        def zb(i, carry):
            for w in range(WREG):
                zbuf[i, pl.ds(w * 16, 16)] = zero16
            return carry

        lax.fori_loop(0, 64, zb, 0)

        def zacc(t, carry):
            pltpu.sync_copy(zbuf, acc.at[pl.ds(rbase + t * 64, 64)])
            return carry

        lax.fori_loop(0, RT // 64, zacc, 0)
        plsc.subcore_barrier()

        for p in range(P):
            xoff = (c * N) if x_shared else ((c * P + p) * N)

            pltpu.sync_copy(src_hbm.at[pl.ds(p * E + ebase, EPT)], srcbuf)

            def addoff(g, carry):
                sl = pl.ds(g * 16, 16)
                srcbuf[sl] = srcbuf[sl] + xoff
                return carry

            lax.fori_loop(0, EPT // 16, addoff, 0)

            def do_chunks(i0, nb, first):
                for b in range(nb):
                    @pl.when(jnp.logical_not(first))
                    def _(b=b):
                        pltpu.make_async_copy(rows[b], acc.at[dstv[b]],
                                              sems[b]).wait()
                    base = p * E + ebase + (i0 + b) * K
                    pltpu.async_copy(dst_hbm.at[pl.ds(base, K)],
                                     dstv[b], semi[b])
                    pltpu.async_copy(vals_hbm.at[pl.ds(base, K)],
                                     valv[b], semi[b])
                gd = []
                for b in range(nb):
                    gd.append(pltpu.async_copy(
                        x_hbm.at[srcbuf.at[pl.ds((i0 + b) * K, K)]],
                        rows[b], semg[b]))
                for b in range(nb):
                    gd[b].wait()
                    pltpu.make_async_copy(dst_hbm.at[pl.ds(0, K)], dstv[b],
                                          semi[b]).wait()
                    pltpu.make_async_copy(vals_hbm.at[pl.ds(0, K)], valv[b],
                                          semi[b]).wait()

                    def scale(g, carry2, b=b):
                        vv = valv[b][pl.ds(g * 16, 16)]
                        for j2 in range(16):
                            v = vv[j2]
                            e = g * 16 + j2
                            for w in range(WREG):
                                sl = pl.ds(w * 16, 16)
                                rows[b][e, sl] = rows[b][e, sl] * v
                        return carry2

                    lax.fori_loop(0, K // 16, scale, 0)
                    pltpu.async_copy(rows[b], acc.at[dstv[b]],
                                     sems[b], add=True)

            def ring(q, carry):
                do_chunks(NB * q, NB, q == 0)
                return carry

            lax.fori_loop(0, CH // NB, ring, 0)
            for b in range(NB):
                pltpu.make_async_copy(rows[b], acc.at[dstv[b]],
                                      sems[b]).wait()
            if TK:
                base = p * E + ebase + CH * K
                pltpu.async_copy(dst_hbm.at[pl.ds(base, TK)], tdst, semi[0])
                pltpu.async_copy(vals_hbm.at[pl.ds(base, TK)], tval, semi[0])
                pltpu.async_copy(x_hbm.at[srcbuf.at[pl.ds(CH * K, TK)]],
                                 trows, semg[0]).wait()
                pltpu.make_async_copy(dst_hbm.at[pl.ds(0, TK)], tdst,
                                      semi[0]).wait()
                pltpu.make_async_copy(vals_hbm.at[pl.ds(0, TK)], tval,
                                      semi[0]).wait()

                def tscale(g, carry2):
                    vv = tval[pl.ds(g * 16, 16)]
                    for j2 in range(16):
                        v = vv[j2]
                        e = g * 16 + j2
                        for w in range(WREG):
                            sl = pl.ds(w * 16, 16)
                            trows[e, sl] = trows[e, sl] * v
                    return carry2

                lax.fori_loop(0, TK // 16, tscale, 0)
                pltpu.sync_copy(trows, acc.at[tdst], add=True)

            if not fuse:
                plsc.subcore_barrier()
                pltpu.sync_copy(acc.at[pl.ds(rbase, RT)],
                                out_hbm.at[c, p, pl.ds(rbase, RT)])
                if p < P - 1:
                    lax.fori_loop(0, RT // 64, zacc, 0)
                    plsc.subcore_barrier()
        if fuse:
            plsc.subcore_barrier()
            pltpu.sync_copy(acc.at[pl.ds(rbase, RT)],
                            out_hbm.at[c, pl.ds(rbase, RT)])

    return k


_spmm192 = _make_spmm(96, fuse=False, x_shared=False, NB=3)
_spmm128 = _make_spmm(64, fuse=False, x_shared=False, NB=4)
_spmm64 = _make_spmm(32, fuse=True, x_shared=True, NB=4)


def _tc1(feature, w2cat):
    def body(f_ref, w_ref, o_ref):
        res = jnp.dot(f_ref[...], w_ref[0], preferred_element_type=jnp.float32)
        o_ref[0, 0] = res[:, :96]
        o_ref[1, 0] = res[:, 96:]

    return pl.pallas_call(
        body,
        grid=(P,),
        in_specs=[pl.BlockSpec((N, NFEAT), lambda p: (0, 0)),
                  pl.BlockSpec((1, NFEAT, 192), lambda p: (p, 0, 0))],
        out_specs=pl.BlockSpec((NC, 1, N, 96), lambda p: (0, p, 0, 0)),
        out_shape=jax.ShapeDtypeStruct((NC, P, N, 96), jnp.float32),
    )(feature, w2cat)


def _tc2(s2, b1s, w2s, sh1b, sh2w):
    def body(s_ref, b1_ref, w2_ref, shb_ref, shw_ref, o_ref):
        sspec = s_ref[0, 0, :, :64] + b1_ref[0]
        a = jnp.dot(_telu(sspec), w2_ref[0], preferred_element_type=jnp.float32)
        ssh = jnp.concatenate([s_ref[0, 0, :, 64:96], s_ref[1, 0, :, :32]],
                              axis=1) + shb_ref[...]
        b = jnp.dot(_telu(ssh), shw_ref[...], preferred_element_type=jnp.float32)
        o_ref[0, 0] = a
        o_ref[1, 0] = b

    return pl.pallas_call(
        body,
        grid=(P,),
        in_specs=[pl.BlockSpec((NC, 1, N, 96), lambda p: (0, p, 0, 0)),
                  pl.BlockSpec((1, 1, NHID), lambda p: (p, 0, 0)),
                  pl.BlockSpec((1, NHID, OUT), lambda p: (p, 0, 0)),
                  pl.BlockSpec((1, NHID), lambda p: (0, 0)),
                  pl.BlockSpec((NHID, OUT), lambda p: (0, 0))],
        out_specs=pl.BlockSpec((NC, 1, N, OUT), lambda p: (0, p, 0, 0)),
        out_shape=jax.ShapeDtypeStruct((NC, P, N, OUT), jnp.float32),
    )(s2, b1s, w2s, sh1b, sh2w)


def _tc3(s4, b2s, sh2b):
    def body(s_ref, b2_ref, shb_ref, spec_ref, shm_ref, hsh_ref, ps_ref):
        p = pl.program_id(0)
        spec = s_ref[0, 0] + b2_ref[0]
        shm = s_ref[1, 0] + shb_ref[...]
        spec_ref[0] = spec
        shm_ref[0] = shm
        mp = jnp.mean(spec, axis=0)
        mx = jnp.max(spec, axis=0)
        z = jnp.exp(spec - mx[None, :])
        prob = z / jnp.sum(z, axis=0)[None, :]
        ent = -jnp.sum(prob * jnp.log(prob + 1e-06), axis=0)
        ps_ref[0, 0] = jnp.concatenate([mp, mx, ent], axis=-1)

        @pl.when(p == 0)
        def _():
            hsh_ref[...] = shm

        @pl.when(p > 0)
        def _():
            hsh_ref[...] = hsh_ref[...] + shm

        @pl.when(p == P - 1)
        def _():
            hsh_ref[...] = hsh_ref[...] * (1.0 / P)

    return pl.pallas_call(
        body,
        grid=(P,),
        in_specs=[pl.BlockSpec((NC, 1, N, OUT), lambda p: (0, p, 0, 0)),
                  pl.BlockSpec((1, 1, OUT), lambda p: (p, 0, 0)),
                  pl.BlockSpec((1, OUT), lambda p: (0, 0))],
        out_specs=[pl.BlockSpec((1, N, OUT), lambda p: (p, 0, 0)),
                   pl.BlockSpec((1, N, OUT), lambda p: (p, 0, 0)),
                   pl.BlockSpec((N, OUT), lambda p: (0, 0)),
                   pl.BlockSpec((1, 1, 3 * OUT), lambda p: (p, 0, 0))],
        out_shape=[jax.ShapeDtypeStruct((P, N, OUT), jnp.float32),
                   jax.ShapeDtypeStruct((P, N, OUT), jnp.float32),
                   jax.ShapeDtypeStruct((N, OUT), jnp.float32),
                   jax.ShapeDtypeStruct((P, 1, 3 * OUT), jnp.float32)],
    )(s4, b2s, sh2b)


def _tc3b(spec, col1w, col1b, col2w, col2b):
    def body(spec_ref, c1w_ref, c1b_ref, c2w_ref, c2b_ref, hcol_ref, acc_ref):
        p = pl.program_id(0)
        contrib = jnp.dot(spec_ref[0], c1w_ref[0],
                          preferred_element_type=jnp.float32)

        @pl.when(p == 0)
        def _():
            acc_ref[...] = contrib

        @pl.when(p > 0)
        def _():
            acc_ref[...] = acc_ref[...] + contrib

        @pl.when(p == P - 1)
        def _():
            h = jax.nn.relu(acc_ref[...] + c1b_ref[...])
            hcol_ref[...] = jnp.dot(h, c2w_ref[...],
                                    preferred_element_type=jnp.float32) + c2b_ref[...]

    return pl.pallas_call(
        body,
        grid=(P,),
        in_specs=[pl.BlockSpec((1, N, OUT), lambda p: (p, 0, 0)),
                  pl.BlockSpec((1, NHID, OUT), lambda p: (p, 0, 0)),
                  pl.BlockSpec((1, OUT), lambda p: (0, 0)),
                  pl.BlockSpec((NHID, OUT), lambda p: (0, 0)),
                  pl.BlockSpec((1, OUT), lambda p: (0, 0))],
        out_specs=pl.BlockSpec((N, OUT), lambda p: (0, 0)),
        out_shape=jax.ShapeDtypeStruct((N, OUT), jnp.float32),
        scratch_shapes=[pltpu.VMEM((N, NHID), jnp.float32)],
    )(spec, col1w, col1b, col2w, col2b)


def _tc4(spec, r1, wt, wp, raw1b, raw2w):
    def body(spec_ref, r1_ref, wt_ref, wp_ref, r1b_ref, r2w_ref,
             hsp_ref, u1_ref, v2_ref):
        p = pl.program_id(0)

        @pl.when(p == 0)
        def _():
            hsp_ref[...] = spec_ref[0] * wp_ref[0]
            u1_ref[...] = r1_ref[0] * wt_ref[0]

        @pl.when(p > 0)
        def _():
            hsp_ref[...] = hsp_ref[...] + spec_ref[0] * wp_ref[0]
            u1_ref[...] = u1_ref[...] + r1_ref[0] * wt_ref[0]

        @pl.when(p == P - 1)
        def _():
            u1 = u1_ref[...] + r1b_ref[...]
            u1_ref[...] = u1
            v = jnp.dot(u1, r2w_ref[...], preferred_element_type=jnp.float32)
            v2_ref[0] = v[:, :32]
            v2_ref[1] = v[:, 32:]

    return pl.pallas_call(
        body,
        grid=(P,),
        in_specs=[pl.BlockSpec((1, N, OUT), lambda p: (p, 0, 0)),
                  pl.BlockSpec((1, N, OUT), lambda p: (p, 0, 0)),
                  pl.BlockSpec((1, 1, 1), lambda p: (p, 0, 0)),
                  pl.BlockSpec((1, 1, 1), lambda p: (p, 0, 0)),
                  pl.BlockSpec((1, OUT), lambda p: (0, 0)),
                  pl.BlockSpec((NHID, OUT), lambda p: (0, 0))],
        out_specs=[pl.BlockSpec((N, OUT), lambda p: (0, 0)),
                   pl.BlockSpec((N, OUT), lambda p: (0, 0)),
                   pl.BlockSpec((NC, N, 32), lambda p: (0, 0, 0))],
        out_shape=[jax.ShapeDtypeStruct((N, OUT), jnp.float32),
                   jax.ShapeDtypeStruct((N, OUT), jnp.float32),
                   jax.ShapeDtypeStruct((NC, N, 32), jnp.float32)],
    )(spec, r1, wt, wp, raw1b, raw2w)


def _tc4b(vals3, wt):
    def body(v_ref, wt_ref, o_ref):
        o_ref[...] = v_ref[...] * wt_ref[...]

    return pl.pallas_call(
        body,
        grid=(P,),
        in_specs=[pl.BlockSpec((1, E // 128, 128), lambda p: (p, 0, 0)),
                  pl.BlockSpec((1, 1, 1), lambda p: (p, 0, 0))],
        out_specs=pl.BlockSpec((1, E // 128, 128), lambda p: (p, 0, 0)),
        out_shape=jax.ShapeDtypeStruct((P, E // 128, 128), jnp.float32),
    )(vals3, wt)


def _tc5(hsp, hsh, hcol, u1, s6, raw2b, projw, projb):
    def body(hsp_ref, hsh_ref, hcol_ref, u1_ref, s6_ref, r2b_ref,
             pw_ref, pb_ref, out_ref, hraw_ref):
        u2 = jnp.concatenate([s6_ref[0], s6_ref[1]], axis=1) + r2b_ref[...]
        hraw = (u1_ref[...] + u2) * 0.5
        hraw_ref[...] = hraw
        all_feat = jnp.concatenate(
            [hsp_ref[...], hsh_ref[...], hcol_ref[...], hraw], axis=1)
        out_ref[...] = jnp.dot(all_feat, pw_ref[...],
                               preferred_element_type=jnp.float32) + pb_ref[...]

    return pl.pallas_call(
        body,
        out_shape=[jax.ShapeDtypeStruct((N, OUT), jnp.float32),
                   jax.ShapeDtypeStruct((N, OUT), jnp.float32)],
    )(hsp, hsh, hcol, u1, s6, raw2b, projw, projb)


def kernel(feature, edge_index_0, edge_index_1, edge_index_2,
           vals_0, vals_1, vals_2, params):
    src_all = jnp.concatenate([edge_index_0[1], edge_index_1[1], edge_index_2[1]])
    dst_all = jnp.concatenate(
        [edge_index_0[0], edge_index_1[0], edge_index_2[0]])
    vals_all = jnp.stack([vals_0, vals_1, vals_2])
    vals_flat = vals_all.reshape(P * E)

    w2cat = jnp.stack([
        jnp.concatenate([params["spec1_W_" + str(p)], params["sh1_W"],
                         params["raw1_W"]], axis=1)
        for p in range(P)])
    x2 = _tc1(feature, w2cat)

    s2 = _spmm192(x2.reshape(NC * P * N, 96), src_all, dst_all,
                  vals_flat)[:, :, :N]

    b1s = jnp.stack([params["spec1_b_" + str(p)] for p in range(P)]).reshape(P, 1, NHID)
    w2s = jnp.stack([params["spec2_W_" + str(p)] for p in range(P)])
    x4 = _tc2(s2, b1s, w2s, params["sh1_b"].reshape(1, NHID), params["sh2_W"])

    s4 = _spmm128(x4.reshape(NC * P * N, 64), src_all, dst_all,
                  vals_flat)[:, :, :N]

    b2s = jnp.stack([params["spec2_b_" + str(p)] for p in range(P)]).reshape(P, 1, OUT)
    spec, shm, hsh, ps = _tc3(s4, b2s, params["sh2_b"].reshape(1, OUT))
    hcol = _tc3b(spec, params["col1_W"].reshape(P, NHID, OUT),
                 params["col1_b"].reshape(1, OUT), params["col2_W"],
                 params["col2_b"].reshape(1, OUT))

    ps = ps.reshape(P, 3 * OUT)
    sim = ps @ ps.T / (np.sqrt(3.0 * OUT) * params["tau"])
    t_mat = jax.nn.softmax(sim, axis=1)
    pi0 = jax.nn.softmax(params["weight_b"].squeeze())
    pi = pi0
    for _ in range(13):
        pi = 0.2 * pi0 + 0.8 * (pi @ t_mat)
    wt = pi.reshape(P, 1, 1)
    wp = jax.nn.softmax(pi).reshape(P, 1, 1)

    r1 = s2[1, :, :, 32:96]
    hsp, u1, v2 = _tc4(spec, r1, wt, wp,
                       params["raw1_b"].reshape(1, OUT),
                       params["raw2_W"])
    vals6 = _tc4b(vals_all.reshape(P, E // 128, 128), wt)

    s6 = _spmm64(v2.reshape(NC * N, 32), src_all, dst_all,
                 vals6.reshape(P * E))[:, :N]

    out, hraw = _tc5(hsp, hsh, hcol, u1, s6,
                     params["raw2_b"].reshape(1, OUT), params["proj_W"],
                     params["proj_b"].reshape(1, OUT))

    return (out, spec[0], spec[1], spec[2], shm[0], shm[1], shm[2],
            hcol, hraw)


# speedup vs baseline: 6.8750x; 1.0011x over previous
import functools

import numpy as np
import jax
import jax.numpy as jnp
from jax import lax
from jax.experimental import pallas as pl
from jax.experimental.pallas import tpu as pltpu
from jax.experimental.pallas import tpu_sc as plsc

N = 10000
E = 320000
NFEAT = 128
NHID = 64
OUT = 64
P = 3

NC = 2
NS = 16
NP_ = 10240
RT = NP_ // NS
EPT = E // NS


def _telu(x):
    return x * jnp.tanh(jnp.exp(x))


def _make_spmm(Wh, fuse, x_shared, K=128, NB=4):
    WREG = Wh // 16
    CH = EPT // K
    TK = EPT - CH * K
    mesh = plsc.VectorSubcoreMesh(core_axis_name="c", subcore_axis_name="s")
    out_type = jax.ShapeDtypeStruct(
        (NC, NP_, Wh) if fuse else (NC, P, NP_, Wh), jnp.float32)
    scratch = [
        pltpu.VMEM((EPT,), jnp.int32),
        [pltpu.VMEM((K,), jnp.int32)] * NB,
        [pltpu.VMEM((K,), jnp.float32)] * NB,
        [pltpu.VMEM((K, Wh), jnp.float32)] * NB,
        pltpu.VMEM((TK,), jnp.int32),
        pltpu.VMEM((TK,), jnp.float32),
        pltpu.VMEM((TK, Wh), jnp.float32),
        pltpu.VMEM((64, Wh), jnp.float32),
        pltpu.VMEM_SHARED((NP_, Wh), jnp.float32),
        [pltpu.SemaphoreType.DMA] * NB,
        [pltpu.SemaphoreType.DMA] * NB,
        [pltpu.SemaphoreType.DMA] * NB,
    ]

    @functools.partial(pl.kernel, out_type=out_type, mesh=mesh,
                       scratch_types=scratch,
                       compiler_params=pltpu.CompilerParams(
                           use_tc_tiling_on_sc=False))
    def k(x_hbm, src_hbm, dst_hbm, vals_hbm, out_hbm,
          srcbuf, dstv, valv, rows, tdst, tval, trows, zbuf, acc,
          semg, sems, semi):
        c = lax.axis_index("c")
        s = lax.axis_index("s")
        ebase = s * EPT
        rbase = s * RT

        zero16 = jnp.zeros((16,), jnp.float32)

        def zb(i, carry):
            for w in range(WREG):
                zbuf[i, pl.ds(w * 16, 16)] = zero16
            return carry

        lax.fori_loop(0, 64, zb, 0)

        def zacc(t, carry):
            pltpu.sync_copy(zbuf, acc.at[pl.ds(rbase + t * 64, 64)])
            return carry

        lax.fori_loop(0, RT // 64, zacc, 0)
        plsc.subcore_barrier()

        for p in range(P):
            xoff = (c * N) if x_shared else ((c * P + p) * N)

            pltpu.sync_copy(src_hbm.at[pl.ds(p * E + ebase, EPT)], srcbuf)

            def addoff(g, carry):
                sl = pl.ds(g * 16, 16)
                srcbuf[sl] = srcbuf[sl] + xoff
                return carry

            lax.fori_loop(0, EPT // 16, addoff, 0)

            def do_chunks(i0, nb, first):
                for b in range(nb):
                    @pl.when(jnp.logical_not(first))
                    def _(b=b):
                        pltpu.make_async_copy(rows[b], acc.at[dstv[b]],
                                              sems[b]).wait()
                    base = p * E + ebase + (i0 + b) * K
                    pltpu.async_copy(dst_hbm.at[pl.ds(base, K)],
                                     dstv[b], semi[b])
                    pltpu.async_copy(vals_hbm.at[pl.ds(base, K)],
                                     valv[b], semi[b])
                gd = []
                for b in range(nb):
                    gd.append(pltpu.async_copy(
                        x_hbm.at[srcbuf.at[pl.ds((i0 + b) * K, K)]],
                        rows[b], semg[b]))
                for b in range(nb):
                    gd[b].wait()
                    pltpu.make_async_copy(dst_hbm.at[pl.ds(0, K)], dstv[b],
                                          semi[b]).wait()
                    pltpu.make_async_copy(vals_hbm.at[pl.ds(0, K)], valv[b],
                                          semi[b]).wait()

                    def scale(g, carry2, b=b):
                        vv = valv[b][pl.ds(g * 16, 16)]
                        for j2 in range(16):
                            v = vv[j2]
                            e = g * 16 + j2
                            for w in range(WREG):
                                sl = pl.ds(w * 16, 16)
                                rows[b][e, sl] = rows[b][e, sl] * v
                        return carry2

                    lax.fori_loop(0, K // 16, scale, 0)
                    pltpu.async_copy(rows[b], acc.at[dstv[b]],
                                     sems[b], add=True)

            def ring(q, carry):
                do_chunks(NB * q, NB, q == 0)
                return carry

            lax.fori_loop(0, CH // NB, ring, 0)
            for b in range(NB):
                pltpu.make_async_copy(rows[b], acc.at[dstv[b]],
                                      sems[b]).wait()
            if TK:
                base = p * E + ebase + CH * K
                pltpu.async_copy(dst_hbm.at[pl.ds(base, TK)], tdst, semi[0])
                pltpu.async_copy(vals_hbm.at[pl.ds(base, TK)], tval, semi[0])
                pltpu.async_copy(x_hbm.at[srcbuf.at[pl.ds(CH * K, TK)]],
                                 trows, semg[0]).wait()
                pltpu.make_async_copy(dst_hbm.at[pl.ds(0, TK)], tdst,
                                      semi[0]).wait()
                pltpu.make_async_copy(vals_hbm.at[pl.ds(0, TK)], tval,
                                      semi[0]).wait()

                def tscale(g, carry2):
                    vv = tval[pl.ds(g * 16, 16)]
                    for j2 in range(16):
                        v = vv[j2]
                        e = g * 16 + j2
                        for w in range(WREG):
                            sl = pl.ds(w * 16, 16)
                            trows[e, sl] = trows[e, sl] * v
                    return carry2

                lax.fori_loop(0, TK // 16, tscale, 0)
                pltpu.sync_copy(trows, acc.at[tdst], add=True)

            if not fuse:
                plsc.subcore_barrier()
                pltpu.sync_copy(acc.at[pl.ds(rbase, RT)],
                                out_hbm.at[c, p, pl.ds(rbase, RT)])
                if p < P - 1:
                    lax.fori_loop(0, RT // 64, zacc, 0)
                    plsc.subcore_barrier()
        if fuse:
            plsc.subcore_barrier()
            pltpu.sync_copy(acc.at[pl.ds(rbase, RT)],
                            out_hbm.at[c, pl.ds(rbase, RT)])

    return k


_spmm192 = _make_spmm(96, fuse=False, x_shared=False, NB=3)
_spmm128 = _make_spmm(64, fuse=False, x_shared=False, NB=4)
_spmm64 = _make_spmm(32, fuse=True, x_shared=True, NB=4)


def _tc1(feature, w2cat):
    def body(f_ref, w_ref, o_ref):
        res = jnp.dot(f_ref[...], w_ref[0], preferred_element_type=jnp.float32)
        o_ref[0, 0] = res[:, :96]
        o_ref[1, 0] = res[:, 96:]

    return pl.pallas_call(
        body,
        grid=(P,),
        in_specs=[pl.BlockSpec((N, NFEAT), lambda p: (0, 0)),
                  pl.BlockSpec((1, NFEAT, 192), lambda p: (p, 0, 0))],
        out_specs=pl.BlockSpec((NC, 1, N, 96), lambda p: (0, p, 0, 0)),
        out_shape=jax.ShapeDtypeStruct((NC, P, N, 96), jnp.float32),
    )(feature, w2cat)


def _tc2(s2, b1s, w2s, sh1b, sh2w):
    def body(s_ref, b1_ref, w2_ref, shb_ref, shw_ref, o_ref):
        sspec = s_ref[0, 0, :, :64] + b1_ref[0]
        a = jnp.dot(_telu(sspec), w2_ref[0], preferred_element_type=jnp.float32)
        ssh = jnp.concatenate([s_ref[0, 0, :, 64:96], s_ref[1, 0, :, :32]],
                              axis=1) + shb_ref[...]
        b = jnp.dot(_telu(ssh), shw_ref[...], preferred_element_type=jnp.float32)
        o_ref[0, 0] = a
        o_ref[1, 0] = b

    return pl.pallas_call(
        body,
        grid=(P,),
        in_specs=[pl.BlockSpec((NC, 1, N, 96), lambda p: (0, p, 0, 0)),
                  pl.BlockSpec((1, 1, NHID), lambda p: (p, 0, 0)),
                  pl.BlockSpec((1, NHID, OUT), lambda p: (p, 0, 0)),
                  pl.BlockSpec((1, NHID), lambda p: (0, 0)),
                  pl.BlockSpec((NHID, OUT), lambda p: (0, 0))],
        out_specs=pl.BlockSpec((NC, 1, N, OUT), lambda p: (0, p, 0, 0)),
        out_shape=jax.ShapeDtypeStruct((NC, P, N, OUT), jnp.float32),
    )(s2, b1s, w2s, sh1b, sh2w)


def _tc3(s4, b2s, sh2b):
    def body(s_ref, b2_ref, shb_ref, spec_ref, shm_ref, hsh_ref, ps_ref):
        p = pl.program_id(0)
        spec = s_ref[0, 0] + b2_ref[0]
        shm = s_ref[1, 0] + shb_ref[...]
        spec_ref[0] = spec
        shm_ref[0] = shm
        mp = jnp.mean(spec, axis=0)
        mx = jnp.max(spec, axis=0)
        z = jnp.exp(spec - mx[None, :])
        prob = z / jnp.sum(z, axis=0)[None, :]
        ent = -jnp.sum(prob * jnp.log(prob + 1e-06), axis=0)
        ps_ref[0, 0] = jnp.concatenate([mp, mx, ent], axis=-1)

        @pl.when(p == 0)
        def _():
            hsh_ref[...] = shm

        @pl.when(p > 0)
        def _():
            hsh_ref[...] = hsh_ref[...] + shm

        @pl.when(p == P - 1)
        def _():
            hsh_ref[...] = hsh_ref[...] * (1.0 / P)

    return pl.pallas_call(
        body,
        grid=(P,),
        in_specs=[pl.BlockSpec((NC, 1, N, OUT), lambda p: (0, p, 0, 0)),
                  pl.BlockSpec((1, 1, OUT), lambda p: (p, 0, 0)),
                  pl.BlockSpec((1, OUT), lambda p: (0, 0))],
        out_specs=[pl.BlockSpec((1, N, OUT), lambda p: (p, 0, 0)),
                   pl.BlockSpec((1, N, OUT), lambda p: (p, 0, 0)),
                   pl.BlockSpec((N, OUT), lambda p: (0, 0)),
                   pl.BlockSpec((1, 1, 3 * OUT), lambda p: (p, 0, 0))],
        out_shape=[jax.ShapeDtypeStruct((P, N, OUT), jnp.float32),
                   jax.ShapeDtypeStruct((P, N, OUT), jnp.float32),
                   jax.ShapeDtypeStruct((N, OUT), jnp.float32),
                   jax.ShapeDtypeStruct((P, 1, 3 * OUT), jnp.float32)],
    )(s4, b2s, sh2b)


def _tc4(spec, r1, ps, wb, tau, col1w, col1b, col2w, col2b,
         raw1b, raw2w):
    def body(spec_ref, r1_ref, ps_ref, wb_ref, tau_ref,
             c1w_ref, c1b_ref, c2w_ref, c2b_ref, r1b_ref, r2w_ref,
             hsp_ref, u1_ref, v2_ref, wt_ref, hcol_ref):
        p = pl.program_id(0)
        psm = ps_ref[...].reshape(P, 3 * OUT)
        sim = jnp.dot(psm, psm.T, preferred_element_type=jnp.float32) / (
            np.sqrt(3.0 * OUT) * tau_ref[0, 0])
        ex = jnp.exp(sim - jnp.max(sim, axis=1, keepdims=True))
        t_mat = ex / jnp.sum(ex, axis=1, keepdims=True)
        ew = jnp.exp(wb_ref[...] - jnp.max(wb_ref[...], axis=1, keepdims=True))
        pi0 = ew / jnp.sum(ew, axis=1, keepdims=True)
        pi = pi0
        for _ in range(13):
            pi = 0.2 * pi0 + 0.8 * jnp.dot(pi, t_mat,
                                           preferred_element_type=jnp.float32)
        ep = jnp.exp(pi - jnp.max(pi, axis=1, keepdims=True))
        wp = ep / jnp.sum(ep, axis=1, keepdims=True)
        iota = lax.broadcasted_iota(jnp.int32, (1, P), 1)
        wt_p = jnp.sum(jnp.where(iota == p, pi, 0.0))
        wp_p = jnp.sum(jnp.where(iota == p, wp, 0.0))
        sp = spec_ref[0]
        contrib = jnp.dot(sp, c1w_ref[0], preferred_element_type=jnp.float32)

        @pl.when(p == 0)
        def _():
            hsp_ref[...] = sp * wp_p
            u1_ref[...] = r1_ref[0] * wt_p
            hcol_ref[...] = contrib

        @pl.when(p > 0)
        def _():
            hsp_ref[...] = hsp_ref[...] + sp * wp_p
            u1_ref[...] = u1_ref[...] + r1_ref[0] * wt_p
            hcol_ref[...] = hcol_ref[...] + contrib

        @pl.when(p == P - 1)
        def _():
            u1 = u1_ref[...] + r1b_ref[...]
            u1_ref[...] = u1
            v = jnp.dot(u1, r2w_ref[...], preferred_element_type=jnp.float32)
            v2_ref[0] = v[:, :32]
            v2_ref[1] = v[:, 32:]
            h = jax.nn.relu(hcol_ref[...] + c1b_ref[...])
            hcol_ref[...] = jnp.dot(h, c2w_ref[...],
                                    preferred_element_type=jnp.float32) + c2b_ref[...]
            wt_ref[...] = pi.reshape(P, 1, 1)

    return pl.pallas_call(
        body,
        grid=(P,),
        in_specs=[pl.BlockSpec((1, N, OUT), lambda p: (p, 0, 0)),
                  pl.BlockSpec((1, N, OUT), lambda p: (p, 0, 0)),
                  pl.BlockSpec((P, 1, 3 * OUT), lambda p: (0, 0, 0)),
                  pl.BlockSpec((1, P), lambda p: (0, 0)),
                  pl.BlockSpec((1, 1), lambda p: (0, 0)),
                  pl.BlockSpec((1, NHID, OUT), lambda p: (p, 0, 0)),
                  pl.BlockSpec((1, OUT), lambda p: (0, 0)),
                  pl.BlockSpec((NHID, OUT), lambda p: (0, 0)),
                  pl.BlockSpec((1, OUT), lambda p: (0, 0)),
                  pl.BlockSpec((1, OUT), lambda p: (0, 0)),
                  pl.BlockSpec((NHID, OUT), lambda p: (0, 0))],
        out_specs=[pl.BlockSpec((N, OUT), lambda p: (0, 0)),
                   pl.BlockSpec((N, OUT), lambda p: (0, 0)),
                   pl.BlockSpec((NC, N, 32), lambda p: (0, 0, 0)),
                   pl.BlockSpec((P, 1, 1), lambda p: (0, 0, 0)),
                   pl.BlockSpec((N, OUT), lambda p: (0, 0))],
        out_shape=[jax.ShapeDtypeStruct((N, OUT), jnp.float32),
                   jax.ShapeDtypeStruct((N, OUT), jnp.float32),
                   jax.ShapeDtypeStruct((NC, N, 32), jnp.float32),
                   jax.ShapeDtypeStruct((P, 1, 1), jnp.float32),
                   jax.ShapeDtypeStruct((N, OUT), jnp.float32)],
    )(spec, r1, ps, wb, tau, col1w, col1b, col2w, col2b,
      raw1b, raw2w)


def _tc4b(vals3, wt):
    def body(v_ref, wt_ref, o_ref):
        o_ref[...] = v_ref[...] * wt_ref[...]

    return pl.pallas_call(
        body,
        grid=(P,),
        in_specs=[pl.BlockSpec((1, E // 128, 128), lambda p: (p, 0, 0)),
                  pl.BlockSpec((1, 1, 1), lambda p: (p, 0, 0))],
        out_specs=pl.BlockSpec((1, E // 128, 128), lambda p: (p, 0, 0)),
        out_shape=jax.ShapeDtypeStruct((P, E // 128, 128), jnp.float32),
    )(vals3, wt)


def _tc5(hsp, hsh, hcol, u1, s6, raw2b, projw, projb):
    def body(hsp_ref, hsh_ref, hcol_ref, u1_ref, s6_ref, r2b_ref,
             pw_ref, pb_ref, out_ref, hraw_ref):
        u2 = jnp.concatenate([s6_ref[0], s6_ref[1]], axis=1) + r2b_ref[...]
        hraw = (u1_ref[...] + u2) * 0.5
        hraw_ref[...] = hraw
        all_feat = jnp.concatenate(
            [hsp_ref[...], hsh_ref[...], hcol_ref[...], hraw], axis=1)
        out_ref[...] = jnp.dot(all_feat, pw_ref[...],
                               preferred_element_type=jnp.float32) + pb_ref[...]

    return pl.pallas_call(
        body,
        out_shape=[jax.ShapeDtypeStruct((N, OUT), jnp.float32),
                   jax.ShapeDtypeStruct((N, OUT), jnp.float32)],
    )(hsp, hsh, hcol, u1, s6, raw2b, projw, projb)


def kernel(feature, edge_index_0, edge_index_1, edge_index_2,
           vals_0, vals_1, vals_2, params):
    src_all = jnp.concatenate([edge_index_0[1], edge_index_1[1], edge_index_2[1]])
    dst_all = jnp.concatenate(
        [edge_index_0[0], edge_index_1[0], edge_index_2[0]])
    vals_all = jnp.stack([vals_0, vals_1, vals_2])
    vals_flat = vals_all.reshape(P * E)

    w2cat = jnp.stack([
        jnp.concatenate([params["spec1_W_" + str(p)], params["sh1_W"],
                         params["raw1_W"]], axis=1)
        for p in range(P)])
    x2 = _tc1(feature, w2cat)

    s2 = _spmm192(x2.reshape(NC * P * N, 96), src_all, dst_all,
                  vals_flat)[:, :, :N]

    b1s = jnp.stack([params["spec1_b_" + str(p)] for p in range(P)]).reshape(P, 1, NHID)
    w2s = jnp.stack([params["spec2_W_" + str(p)] for p in range(P)])
    x4 = _tc2(s2, b1s, w2s, params["sh1_b"].reshape(1, NHID), params["sh2_W"])

    s4 = _spmm128(x4.reshape(NC * P * N, 64), src_all, dst_all,
                  vals_flat)[:, :, :N]

    b2s = jnp.stack([params["spec2_b_" + str(p)] for p in range(P)]).reshape(P, 1, OUT)
    spec, shm, hsh, ps = _tc3(s4, b2s, params["sh2_b"].reshape(1, OUT))

    r1 = s2[1, :, :, 32:96]
    hsp, u1, v2, wt3, hcol = _tc4(
        spec, r1, ps,
        params["weight_b"].reshape(1, P), params["tau"].reshape(1, 1),
        params["col1_W"].reshape(P, NHID, OUT),
        params["col1_b"].reshape(1, OUT), params["col2_W"],
        params["col2_b"].reshape(1, OUT),
        params["raw1_b"].reshape(1, OUT), params["raw2_W"])
    vals6 = _tc4b(vals_all.reshape(P, E // 128, 128), wt3)

    s6 = _spmm64(v2.reshape(NC * N, 32), src_all, dst_all,
                 vals6.reshape(P * E))[:, :N]

    out, hraw = _tc5(hsp, hsh, hcol, u1, s6,
                     params["raw2_b"].reshape(1, OUT), params["proj_W"],
                     params["proj_b"].reshape(1, OUT))

    return (out, spec[0], spec[1], spec[2], shm[0], shm[1], shm[2],
            hcol, hraw)


# speedup vs baseline: 7.0307x; 1.0226x over previous
import functools

import numpy as np
import jax
import jax.numpy as jnp
from jax import lax
from jax.experimental import pallas as pl
from jax.experimental.pallas import tpu as pltpu
from jax.experimental.pallas import tpu_sc as plsc

N = 10000
E = 320000
NFEAT = 128
NHID = 64
OUT = 64
P = 3

NC = 2
NS = 16
NP_ = 10240
RT = NP_ // NS
EPT = E // NS


def _telu(x):
    return x * jnp.tanh(jnp.exp(x))


def _make_spmm(Wh, fuse, x_shared, K=128, NB=4, stream_src=False):
    WREG = Wh // 16
    CH = EPT // K
    TK = EPT - CH * K
    mesh = plsc.VectorSubcoreMesh(core_axis_name="c", subcore_axis_name="s")
    out_type = jax.ShapeDtypeStruct(
        (NC, NP_, Wh) if fuse else (NC, P, NP_, Wh), jnp.float32)
    scratch = [
        ([pltpu.VMEM((K,), jnp.int32)] * NB) if stream_src
        else pltpu.VMEM((EPT,), jnp.int32),
        [pltpu.VMEM((K,), jnp.int32)] * NB,
        [pltpu.VMEM((K,), jnp.float32)] * NB,
        [pltpu.VMEM((K, Wh), jnp.float32)] * NB,
        pltpu.VMEM((TK,), jnp.int32),
        pltpu.VMEM((TK,), jnp.int32),
        pltpu.VMEM((TK,), jnp.float32),
        pltpu.VMEM((TK, Wh), jnp.float32),
        pltpu.VMEM((64, Wh), jnp.float32),
        pltpu.VMEM_SHARED((NP_, Wh), jnp.float32),
        [pltpu.SemaphoreType.DMA] * NB,
        [pltpu.SemaphoreType.DMA] * NB,
        [pltpu.SemaphoreType.DMA] * NB,
        [pltpu.SemaphoreType.DMA] * NB,
    ]

    @functools.partial(pl.kernel, out_type=out_type, mesh=mesh,
                       scratch_types=scratch,
                       compiler_params=pltpu.CompilerParams(
                           use_tc_tiling_on_sc=False))
    def k(x_hbm, src_hbm, dst_hbm, vals_hbm, out_hbm,
          srcst, dstv, valv, rows, tdst, tsrc, tval, trows, zbuf, acc,
          semg, sems, semi, semr):
        c = lax.axis_index("c")
        s = lax.axis_index("s")
        ebase = s * EPT
        rbase = s * RT

        zero16 = jnp.zeros((16,), jnp.float32)

        def zb(i, carry):
            for w in range(WREG):
                zbuf[i, pl.ds(w * 16, 16)] = zero16
            return carry

        lax.fori_loop(0, 64, zb, 0)

        def zacc(t, carry):
            pltpu.sync_copy(zbuf, acc.at[pl.ds(rbase + t * 64, 64)])
            return carry

        lax.fori_loop(0, RT // 64, zacc, 0)
        plsc.subcore_barrier()

        for p in range(P):
            xoff = (c * N) if x_shared else ((c * P + p) * N)

            if not stream_src:
                pltpu.sync_copy(src_hbm.at[pl.ds(p * E + ebase, EPT)], srcst)

                def addoff(g, carry):
                    sl = pl.ds(g * 16, 16)
                    srcst[sl] = srcst[sl] + xoff
                    return carry

                lax.fori_loop(0, EPT // 16, addoff, 0)

            def do_chunks(i0, nb, first):
                for b in range(nb):
                    @pl.when(jnp.logical_not(first))
                    def _(b=b):
                        pltpu.make_async_copy(rows[b], acc.at[dstv[b]],
                                              sems[b]).wait()
                    base = p * E + ebase + (i0 + b) * K
                    pltpu.async_copy(dst_hbm.at[pl.ds(base, K)],
                                     dstv[b], semi[b])
                    pltpu.async_copy(vals_hbm.at[pl.ds(base, K)],
                                     valv[b], semi[b])
                    if stream_src:
                        pltpu.async_copy(src_hbm.at[pl.ds(base, K)],
                                         srcst[b], semr[b])
                gd = []
                for b in range(nb):
                    if stream_src:
                        pltpu.make_async_copy(src_hbm.at[pl.ds(0, K)],
                                              srcst[b], semr[b]).wait()
                        for w in range(K // 16):
                            sl = pl.ds(w * 16, 16)
                            srcst[b][sl] = srcst[b][sl] + xoff
                        gd.append(pltpu.async_copy(
                            x_hbm.at[srcst[b]], rows[b], semg[b]))
                    else:
                        gd.append(pltpu.async_copy(
                            x_hbm.at[srcst.at[pl.ds((i0 + b) * K, K)]],
                            rows[b], semg[b]))
                for b in range(nb):
                    gd[b].wait()
                    pltpu.make_async_copy(dst_hbm.at[pl.ds(0, K)], dstv[b],
                                          semi[b]).wait()
                    pltpu.make_async_copy(vals_hbm.at[pl.ds(0, K)], valv[b],
                                          semi[b]).wait()

                    def scale(g, carry2, b=b):
                        vv = valv[b][pl.ds(g * 16, 16)]
                        for j2 in range(16):
                            v = vv[j2]
                            e = g * 16 + j2
                            for w in range(WREG):
                                sl = pl.ds(w * 16, 16)
                                rows[b][e, sl] = rows[b][e, sl] * v
                        return carry2

                    lax.fori_loop(0, K // 16, scale, 0)
                    pltpu.async_copy(rows[b], acc.at[dstv[b]],
                                     sems[b], add=True)

            def ring(q, carry):
                do_chunks(NB * q, NB, q == 0)
                return carry

            lax.fori_loop(0, CH // NB, ring, 0)
            for b in range(NB):
                pltpu.make_async_copy(rows[b], acc.at[dstv[b]],
                                      sems[b]).wait()
            if TK:
                base = p * E + ebase + CH * K
                pltpu.async_copy(dst_hbm.at[pl.ds(base, TK)], tdst, semi[0])
                pltpu.async_copy(vals_hbm.at[pl.ds(base, TK)], tval, semi[0])
                if stream_src:
                    pltpu.async_copy(src_hbm.at[pl.ds(base, TK)], tsrc,
                                     semr[0]).wait()
                    for w in range(TK // 16):
                        sl = pl.ds(w * 16, 16)
                        tsrc[sl] = tsrc[sl] + xoff
                    pltpu.async_copy(x_hbm.at[tsrc], trows, semg[0]).wait()
                else:
                    pltpu.async_copy(x_hbm.at[srcst.at[pl.ds(CH * K, TK)]],
                                     trows, semg[0]).wait()
                pltpu.make_async_copy(dst_hbm.at[pl.ds(0, TK)], tdst,
                                      semi[0]).wait()
                pltpu.make_async_copy(vals_hbm.at[pl.ds(0, TK)], tval,
                                      semi[0]).wait()

                def tscale(g, carry2):
                    vv = tval[pl.ds(g * 16, 16)]
                    for j2 in range(16):
                        v = vv[j2]
                        e = g * 16 + j2
                        for w in range(WREG):
                            sl = pl.ds(w * 16, 16)
                            trows[e, sl] = trows[e, sl] * v
                    return carry2

                lax.fori_loop(0, TK // 16, tscale, 0)
                pltpu.sync_copy(trows, acc.at[tdst], add=True)

            if not fuse:
                plsc.subcore_barrier()
                pltpu.sync_copy(acc.at[pl.ds(rbase, RT)],
                                out_hbm.at[c, p, pl.ds(rbase, RT)])
                if p < P - 1:
                    lax.fori_loop(0, RT // 64, zacc, 0)
                    plsc.subcore_barrier()
        if fuse:
            plsc.subcore_barrier()
            pltpu.sync_copy(acc.at[pl.ds(rbase, RT)],
                            out_hbm.at[c, pl.ds(rbase, RT)])

    return k


_spmm192 = _make_spmm(96, fuse=False, x_shared=False, NB=4,
                      stream_src=True)
_spmm128 = _make_spmm(64, fuse=False, x_shared=False, NB=4)
_spmm64 = _make_spmm(32, fuse=True, x_shared=True, NB=4)


def _tc1(feature, w2cat):
    def body(f_ref, w_ref, o_ref):
        res = jnp.dot(f_ref[...], w_ref[0], preferred_element_type=jnp.float32)
        o_ref[0, 0] = res[:, :96]
        o_ref[1, 0] = res[:, 96:]

    return pl.pallas_call(
        body,
        grid=(P,),
        in_specs=[pl.BlockSpec((N, NFEAT), lambda p: (0, 0)),
                  pl.BlockSpec((1, NFEAT, 192), lambda p: (p, 0, 0))],
        out_specs=pl.BlockSpec((NC, 1, N, 96), lambda p: (0, p, 0, 0)),
        out_shape=jax.ShapeDtypeStruct((NC, P, N, 96), jnp.float32),
    )(feature, w2cat)


def _tc2(s2, b1s, w2s, sh1b, sh2w):
    def body(s_ref, b1_ref, w2_ref, shb_ref, shw_ref, o_ref):
        sspec = s_ref[0, 0, :, :64] + b1_ref[0]
        a = jnp.dot(_telu(sspec), w2_ref[0], preferred_element_type=jnp.float32)
        ssh = jnp.concatenate([s_ref[0, 0, :, 64:96], s_ref[1, 0, :, :32]],
                              axis=1) + shb_ref[...]
        b = jnp.dot(_telu(ssh), shw_ref[...], preferred_element_type=jnp.float32)
        o_ref[0, 0] = a
        o_ref[1, 0] = b

    return pl.pallas_call(
        body,
        grid=(P,),
        in_specs=[pl.BlockSpec((NC, 1, N, 96), lambda p: (0, p, 0, 0)),
                  pl.BlockSpec((1, 1, NHID), lambda p: (p, 0, 0)),
                  pl.BlockSpec((1, NHID, OUT), lambda p: (p, 0, 0)),
                  pl.BlockSpec((1, NHID), lambda p: (0, 0)),
                  pl.BlockSpec((NHID, OUT), lambda p: (0, 0))],
        out_specs=pl.BlockSpec((NC, 1, N, OUT), lambda p: (0, p, 0, 0)),
        out_shape=jax.ShapeDtypeStruct((NC, P, N, OUT), jnp.float32),
    )(s2, b1s, w2s, sh1b, sh2w)


def _tc3(s4, b2s, sh2b):
    def body(s_ref, b2_ref, shb_ref, spec_ref, shm_ref, hsh_ref, ps_ref):
        p = pl.program_id(0)
        spec = s_ref[0, 0] + b2_ref[0]
        shm = s_ref[1, 0] + shb_ref[...]
        spec_ref[0] = spec
        shm_ref[0] = shm
        mp = jnp.mean(spec, axis=0)
        mx = jnp.max(spec, axis=0)
        z = jnp.exp(spec - mx[None, :])
        prob = z / jnp.sum(z, axis=0)[None, :]
        ent = -jnp.sum(prob * jnp.log(prob + 1e-06), axis=0)
        ps_ref[0, 0] = jnp.concatenate([mp, mx, ent], axis=-1)

        @pl.when(p == 0)
        def _():
            hsh_ref[...] = shm

        @pl.when(p > 0)
        def _():
            hsh_ref[...] = hsh_ref[...] + shm

        @pl.when(p == P - 1)
        def _():
            hsh_ref[...] = hsh_ref[...] * (1.0 / P)

    return pl.pallas_call(
        body,
        grid=(P,),
        in_specs=[pl.BlockSpec((NC, 1, N, OUT), lambda p: (0, p, 0, 0)),
                  pl.BlockSpec((1, 1, OUT), lambda p: (p, 0, 0)),
                  pl.BlockSpec((1, OUT), lambda p: (0, 0))],
        out_specs=[pl.BlockSpec((1, N, OUT), lambda p: (p, 0, 0)),
                   pl.BlockSpec((1, N, OUT), lambda p: (p, 0, 0)),
                   pl.BlockSpec((N, OUT), lambda p: (0, 0)),
                   pl.BlockSpec((1, 1, 3 * OUT), lambda p: (p, 0, 0))],
        out_shape=[jax.ShapeDtypeStruct((P, N, OUT), jnp.float32),
                   jax.ShapeDtypeStruct((P, N, OUT), jnp.float32),
                   jax.ShapeDtypeStruct((N, OUT), jnp.float32),
                   jax.ShapeDtypeStruct((P, 1, 3 * OUT), jnp.float32)],
    )(s4, b2s, sh2b)


def _tc4(spec, r1, ps, wb, tau, col1w, col1b, col2w, col2b,
         raw1b, raw2w):
    def body(spec_ref, r1_ref, ps_ref, wb_ref, tau_ref,
             c1w_ref, c1b_ref, c2w_ref, c2b_ref, r1b_ref, r2w_ref,
             hsp_ref, u1_ref, v2_ref, wt_ref, hcol_ref):
        p = pl.program_id(0)
        psm = ps_ref[...].reshape(P, 3 * OUT)
        sim = jnp.dot(psm, psm.T, preferred_element_type=jnp.float32) / (
            np.sqrt(3.0 * OUT) * tau_ref[0, 0])
        ex = jnp.exp(sim - jnp.max(sim, axis=1, keepdims=True))
        t_mat = ex / jnp.sum(ex, axis=1, keepdims=True)
        ew = jnp.exp(wb_ref[...] - jnp.max(wb_ref[...], axis=1, keepdims=True))
        pi0 = ew / jnp.sum(ew, axis=1, keepdims=True)
        pi = pi0
        for _ in range(13):
            pi = 0.2 * pi0 + 0.8 * jnp.dot(pi, t_mat,
                                           preferred_element_type=jnp.float32)
        ep = jnp.exp(pi - jnp.max(pi, axis=1, keepdims=True))
        wp = ep / jnp.sum(ep, axis=1, keepdims=True)
        iota = lax.broadcasted_iota(jnp.int32, (1, P), 1)
        wt_p = jnp.sum(jnp.where(iota == p, pi, 0.0))
        wp_p = jnp.sum(jnp.where(iota == p, wp, 0.0))
        sp = spec_ref[0]
        contrib = jnp.dot(sp, c1w_ref[0], preferred_element_type=jnp.float32)

        @pl.when(p == 0)
        def _():
            hsp_ref[...] = sp * wp_p
            u1_ref[...] = r1_ref[0] * wt_p
            hcol_ref[...] = contrib

        @pl.when(p > 0)
        def _():
            hsp_ref[...] = hsp_ref[...] + sp * wp_p
            u1_ref[...] = u1_ref[...] + r1_ref[0] * wt_p
            hcol_ref[...] = hcol_ref[...] + contrib

        @pl.when(p == P - 1)
        def _():
            u1 = u1_ref[...] + r1b_ref[...]
            u1_ref[...] = u1
            v = jnp.dot(u1, r2w_ref[...], preferred_element_type=jnp.float32)
            v2_ref[0] = v[:, :32]
            v2_ref[1] = v[:, 32:]
            h = jax.nn.relu(hcol_ref[...] + c1b_ref[...])
            hcol_ref[...] = jnp.dot(h, c2w_ref[...],
                                    preferred_element_type=jnp.float32) + c2b_ref[...]
            wt_ref[...] = pi.reshape(P, 1, 1)

    return pl.pallas_call(
        body,
        grid=(P,),
        in_specs=[pl.BlockSpec((1, N, OUT), lambda p: (p, 0, 0)),
                  pl.BlockSpec((1, N, OUT), lambda p: (p, 0, 0)),
                  pl.BlockSpec((P, 1, 3 * OUT), lambda p: (0, 0, 0)),
                  pl.BlockSpec((1, P), lambda p: (0, 0)),
                  pl.BlockSpec((1, 1), lambda p: (0, 0)),
                  pl.BlockSpec((1, NHID, OUT), lambda p: (p, 0, 0)),
                  pl.BlockSpec((1, OUT), lambda p: (0, 0)),
                  pl.BlockSpec((NHID, OUT), lambda p: (0, 0)),
                  pl.BlockSpec((1, OUT), lambda p: (0, 0)),
                  pl.BlockSpec((1, OUT), lambda p: (0, 0)),
                  pl.BlockSpec((NHID, OUT), lambda p: (0, 0))],
        out_specs=[pl.BlockSpec((N, OUT), lambda p: (0, 0)),
                   pl.BlockSpec((N, OUT), lambda p: (0, 0)),
                   pl.BlockSpec((NC, N, 32), lambda p: (0, 0, 0)),
                   pl.BlockSpec((P, 1, 1), lambda p: (0, 0, 0)),
                   pl.BlockSpec((N, OUT), lambda p: (0, 0))],
        out_shape=[jax.ShapeDtypeStruct((N, OUT), jnp.float32),
                   jax.ShapeDtypeStruct((N, OUT), jnp.float32),
                   jax.ShapeDtypeStruct((NC, N, 32), jnp.float32),
                   jax.ShapeDtypeStruct((P, 1, 1), jnp.float32),
                   jax.ShapeDtypeStruct((N, OUT), jnp.float32)],
    )(spec, r1, ps, wb, tau, col1w, col1b, col2w, col2b,
      raw1b, raw2w)


def _tc4b(vals3, wt):
    def body(v_ref, wt_ref, o_ref):
        o_ref[...] = v_ref[...] * wt_ref[...]

    return pl.pallas_call(
        body,
        grid=(P,),
        in_specs=[pl.BlockSpec((1, E // 128, 128), lambda p: (p, 0, 0)),
                  pl.BlockSpec((1, 1, 1), lambda p: (p, 0, 0))],
        out_specs=pl.BlockSpec((1, E // 128, 128), lambda p: (p, 0, 0)),
        out_shape=jax.ShapeDtypeStruct((P, E // 128, 128), jnp.float32),
    )(vals3, wt)


def _tc5(hsp, hsh, hcol, u1, s6, raw2b, projw, projb):
    def body(hsp_ref, hsh_ref, hcol_ref, u1_ref, s6_ref, r2b_ref,
             pw_ref, pb_ref, out_ref, hraw_ref):
        u2 = jnp.concatenate([s6_ref[0], s6_ref[1]], axis=1) + r2b_ref[...]
        hraw = (u1_ref[...] + u2) * 0.5
        hraw_ref[...] = hraw
        all_feat = jnp.concatenate(
            [hsp_ref[...], hsh_ref[...], hcol_ref[...], hraw], axis=1)
        out_ref[...] = jnp.dot(all_feat, pw_ref[...],
                               preferred_element_type=jnp.float32) + pb_ref[...]

    return pl.pallas_call(
        body,
        out_shape=[jax.ShapeDtypeStruct((N, OUT), jnp.float32),
                   jax.ShapeDtypeStruct((N, OUT), jnp.float32)],
    )(hsp, hsh, hcol, u1, s6, raw2b, projw, projb)


def kernel(feature, edge_index_0, edge_index_1, edge_index_2,
           vals_0, vals_1, vals_2, params):
    src_all = jnp.concatenate([edge_index_0[1], edge_index_1[1], edge_index_2[1]])
    dst_all = jnp.concatenate(
        [edge_index_0[0], edge_index_1[0], edge_index_2[0]])
    vals_all = jnp.stack([vals_0, vals_1, vals_2])
    vals_flat = vals_all.reshape(P * E)

    w2cat = jnp.stack([
        jnp.concatenate([params["spec1_W_" + str(p)], params["sh1_W"],
                         params["raw1_W"]], axis=1)
        for p in range(P)])
    x2 = _tc1(feature, w2cat)

    s2 = _spmm192(x2.reshape(NC * P * N, 96), src_all, dst_all,
                  vals_flat)[:, :, :N]

    b1s = jnp.stack([params["spec1_b_" + str(p)] for p in range(P)]).reshape(P, 1, NHID)
    w2s = jnp.stack([params["spec2_W_" + str(p)] for p in range(P)])
    x4 = _tc2(s2, b1s, w2s, params["sh1_b"].reshape(1, NHID), params["sh2_W"])

    s4 = _spmm128(x4.reshape(NC * P * N, 64), src_all, dst_all,
                  vals_flat)[:, :, :N]

    b2s = jnp.stack([params["spec2_b_" + str(p)] for p in range(P)]).reshape(P, 1, OUT)
    spec, shm, hsh, ps = _tc3(s4, b2s, params["sh2_b"].reshape(1, OUT))

    r1 = s2[1, :, :, 32:96]
    hsp, u1, v2, wt3, hcol = _tc4(
        spec, r1, ps,
        params["weight_b"].reshape(1, P), params["tau"].reshape(1, 1),
        params["col1_W"].reshape(P, NHID, OUT),
        params["col1_b"].reshape(1, OUT), params["col2_W"],
        params["col2_b"].reshape(1, OUT),
        params["raw1_b"].reshape(1, OUT), params["raw2_W"])
    vals6 = _tc4b(vals_all.reshape(P, E // 128, 128), wt3)

    s6 = _spmm64(v2.reshape(NC * N, 32), src_all, dst_all,
                 vals6.reshape(P * E))[:, :N]

    out, hraw = _tc5(hsp, hsh, hcol, u1, s6,
                     params["raw2_b"].reshape(1, OUT), params["proj_W"],
                     params["proj_b"].reshape(1, OUT))

    return (out, spec[0], spec[1], spec[2], shm[0], shm[1], shm[2],
            hcol, hraw)
